# baseline (speedup 1.0000x reference)
import functools
import math

import jax
import jax.numpy as jnp
from jax import lax
from jax.experimental import pallas as pl
from jax.experimental.pallas import tpu as pltpu

D = 1024
BD = 2048
H = 16
DH = 64
BLK = 64
WS = 128
NSACC = 2
TOPK = 16


def _dot(a, b):
    return jnp.dot(a, b, preferred_element_type=jnp.float32)


def _ln_rows(x, g, b, eps=1e-5):
    m = jnp.mean(x, axis=-1, keepdims=True)
    v = jnp.mean((x - m) ** 2, axis=-1, keepdims=True)
    return (x - m) * lax.rsqrt(v + eps) * g + b


def _gelu(x):
    return 0.5 * x * (1.0 + lax.erf(x * (2.0 ** -0.5)))


def _ctrl_compute(state, pm2, cqw, cqb, ckw, ckb, n1g, n1b, fw, fb):
    B = state.shape[0]
    nb = pm2.shape[0] // B
    q = _dot(state, cqw.T) + cqb
    kk = _dot(pm2, ckw.T) + ckb
    sfull = _dot(q, kk.T) / math.sqrt(D)
    rows = [sfull[bi:bi + 1, bi * nb:(bi + 1) * nb] for bi in range(B)]
    scores = jnp.concatenate(rows, axis=0)

    iota = lax.broadcasted_iota(jnp.int32, (B, nb), 1)
    work = scores
    tvs, tis = [], []
    for _ in range(TOPK):
        m = jnp.max(work, axis=1, keepdims=True)
        idx = jnp.min(jnp.where(work == m, iota, nb), axis=1, keepdims=True)
        tvs.append(m)
        tis.append(idx)
        work = jnp.where(iota == idx, -jnp.inf, work)
    tv = jnp.concatenate(tvs, axis=1)
    ti = jnp.concatenate(tis, axis=1)
    e = jnp.exp((tv - tv[:, 0:1]) / 5.0)
    tw = e / jnp.sum(e, axis=1, keepdims=True)

    g = _ln_rows(state, n1g, n1b)
    qf = _dot(g, fw[:D, :].T) + fb[0:1, :D]
    ks = _dot(g, fw[D:2 * D, :].T) + fb[0:1, D:2 * D]
    vs = _dot(g, fw[2 * D:, :].T) + fb[0:1, 2 * D:]
    return scores, ti, tw, qf, ks, vs


def _periph_stage1_kernel(x_ref, wc_ref, bc_ref, xmid_ref, std_ref, max_ref):
    x = x_ref[0]
    xmid_ref[0] = lax.dot_general(
        wc_ref[...], x, (((1,), (1,)), ((), ())),
        preferred_element_type=jnp.float32) + bc_ref[...]
    mean = jnp.mean(x, axis=0, keepdims=True)
    var = jnp.sum((x - mean) ** 2, axis=0, keepdims=True) / (BLK - 1)
    std_ref[0] = jnp.sqrt(var)
    max_ref[0] = jnp.max(x, axis=0, keepdims=True)


def _periph_ctrl_kernel(cin_ref, wf_ref, cb_ref, sv_ref, sw_ref, sb_ref,
                        mv_ref, mw_ref, mb_ref, pp_ref, pb_ref, g_ref, b_ref,
                        pos_ref, cqw_ref, cqb_ref, ckw_ref, ckb_ref, n1g_ref,
                        n1b_ref, fw_ref, fb_ref,
                        pmap_ref, state_ref, scores_ref, ti_ref, tw_ref,
                        qf_ref, ks_ref, vs_ref):
    co = _dot(cin_ref[...], wf_ref[...].T) + cb_ref[...]
    so = _dot(sv_ref[...], sw_ref[...].T) + sb_ref[...]
    mo = _dot(mv_ref[...], mw_ref[...].T) + mb_ref[...]
    combined = jnp.concatenate([co, so, mo], axis=1)
    pre = _dot(combined, pp_ref[...].T) + pb_ref[...]
    pmap = _ln_rows(pre, g_ref[...], b_ref[...]) + pos_ref[...]
    pmap_ref[...] = pmap
    B = state_ref.shape[0]
    nb = pmap.shape[0] // B
    srows = [jnp.mean(pmap[bi * nb:(bi + 1) * nb], axis=0, keepdims=True)
             for bi in range(B)]
    state = jnp.concatenate(srows, axis=0)
    state_ref[...] = state
    scores, ti, tw, qf, ks, vs = _ctrl_compute(
        state, pmap, cqw_ref[...], cqb_ref[...], ckw_ref[...], ckb_ref[...],
        n1g_ref[...], n1b_ref[...], fw_ref[...], fb_ref[...])
    scores_ref[...] = scores
    ti_ref[...] = ti
    tw_ref[...] = tw
    qf_ref[...] = qf
    ks_ref[...] = ks
    vs_ref[...] = vs


def _kv_kernel(x_ref, l1g_ref, l1b_ref, n1g_ref, n1b_ref, fw_ref, fb_ref,
               kt_ref, vt_ref):
    h = _ln_rows(x_ref[...], l1g_ref[...], l1b_ref[...])
    g = _ln_rows(h, n1g_ref[...], n1b_ref[...])
    kt_ref[...] = _dot(g, fw_ref[D:2 * D, :].T) + fb_ref[0:1, D:2 * D]
    vt_ref[...] = _dot(g, fw_ref[2 * D:, :].T) + fb_ref[0:1, 2 * D:]


def _controller_kernel(state_ref, pmap_ref, cqw_ref, cqb_ref, ckw_ref,
                       ckb_ref, n1g_ref, n1b_ref, fw_ref, fb_ref,
                       scores_ref, ti_ref, tw_ref, qf_ref, ks_ref, vs_ref):
    B = state_ref.shape[0]
    nb = pmap_ref.shape[1]
    pm2 = pmap_ref[...].reshape(B * nb, D)
    scores, ti, tw, qf, ks, vs = _ctrl_compute(
        state_ref[...], pm2, cqw_ref[...], cqb_ref[...], ckw_ref[...],
        ckb_ref[...], n1g_ref[...], n1b_ref[...], fw_ref[...], fb_ref[...])
    scores_ref[...] = scores
    ti_ref[...] = ti
    tw_ref[...] = tw
    qf_ref[...] = qf
    ks_ref[...] = ks
    vs_ref[...] = vs


def _foveal_attn_kernel(starts_ref, astarts_ref, kt_ref, vt_ref, qm_ref,
                        ks_ref, vs_ref, ex_ref, ctxv_ref, *, nacc):
    b = pl.program_id(0)
    qm = qm_ref[0]
    ex = ex_ref[...]
    s_state = _dot(ks_ref[0], qm) / 8.0
    vs = vs_ref[0]

    saccs, vaccs = [], []
    for j in range(nacc):
        a0 = astarts_ref[b, j] * 8
        kacc = kt_ref[0, pl.ds(a0, WS), :]
        vacc = vt_ref[0, pl.ds(a0, WS), :]
        saccs.append(_dot(kacc, qm) / 8.0)
        vaccs.append(vacc)

    for k in range(TOPK):
        st = starts_ref[b, k] * 8
        kwin = kt_ref[0, pl.ds(st, WS), :]
        vwin = vt_ref[0, pl.ds(st, WS), :]
        sw = _dot(kwin, qm) / 8.0
        m = jnp.maximum(jnp.max(sw, axis=0, keepdims=True), s_state)
        for sa in saccs:
            m = jnp.maximum(m, jnp.max(sa, axis=0, keepdims=True))
        ew = jnp.exp(sw - m)
        es = jnp.exp(s_state - m)
        denom = jnp.sum(ew, axis=0, keepdims=True) + es
        eas = []
        for sa in saccs:
            ea = jnp.exp(sa - m)
            eas.append(ea)
            denom = denom + jnp.sum(ea, axis=0, keepdims=True)
        inv = 1.0 / denom
        ctxv = jnp.sum(vwin * _dot(ew * inv, ex), axis=0, keepdims=True)
        ctxv = ctxv + vs * _dot(es * inv, ex)
        for ea, vacc in zip(eas, vaccs):
            ctxv = ctxv + jnp.sum(vacc * _dot(ea * inv, ex), axis=0,
                                  keepdims=True)
        ctxv_ref[0, k:k + 1, :] = ctxv


def _epilogue_kernel(astarts_ref, alpha_ref, ctxv_ref, st32_ref, tw_ref,
                     ow_ref, ob_ref, n2g_ref, n2b_ref, w1_ref, b1_ref,
                     w2_ref, b2_ref, pmap_ref, ng_ref, nbg_ref, mw_ref,
                     mb_ref, wo_ref, bo_ref, l1g_ref, l1b_ref, x_hbm,
                     wst_ref, out_ref, accx_ref, sem, *, nacc):
    B, K = tw_ref.shape
    s = st32_ref[...] + _dot(ctxv_ref[...], ow_ref[...].T) + ob_ref[...]
    u = _ln_rows(s, n2g_ref[...], n2b_ref[...])
    m1 = _gelu(_dot(u, w1_ref[...].T) + b1_ref[...])
    s2 = s + _dot(m1, w2_ref[...].T) + b2_ref[...]
    for bi in range(B):
        wst_ref[bi:bi + 1, :] = _dot(tw_ref[bi:bi + 1, :],
                                     s2[bi * K:(bi + 1) * K, :])

    alpha = alpha_ref[0, 0]
    for bi in range(B):
        for j in range(nacc):
            a0 = astarts_ref[bi, j] * 8
            cp = pltpu.make_async_copy(
                x_hbm.at[bi, pl.ds(a0, WS), :],
                accx_ref.at[pl.ds(j * WS, WS), :], sem)
            cp.start()
            cp.wait()
        a = _ln_rows(accx_ref[...], l1g_ref[...], l1b_ref[...])
        pm = pmap_ref[bi]
        pn = _ln_rows(pm, ng_ref[...], nbg_ref[...])
        q = _dot(pn, mw_ref[:D, :].T) + mb_ref[0:1, :D]
        ka = _dot(a, mw_ref[D:2 * D, :].T) + mb_ref[0:1, D:2 * D]
        va = _dot(a, mw_ref[2 * D:, :].T) + mb_ref[0:1, 2 * D:]
        pieces = []
        for hh in range(H):
            sl = slice(hh * DH, (hh + 1) * DH)
            sc = _dot(q[:, sl], ka[:, sl].T) / 8.0
            sc = sc - jnp.max(sc, axis=1, keepdims=True)
            pr = jnp.exp(sc)
            pr = pr / jnp.sum(pr, axis=1, keepdims=True)
            pieces.append(_dot(pr, va[:, sl]))
        ctx = jnp.concatenate(pieces, axis=1)
        delta = _dot(ctx, wo_ref[...].T) + bo_ref[...]
        out_ref[bi] = pm + alpha * delta


def _final_kernel(res_ref, state_ref, og_ref, obn_ref, ow_ref, ob_ref,
                  l2g_ref, l2b_ref, w1_ref, b1_ref, w2_ref, b2_ref, out_ref,
                  w1b_ref, w2b_ref):
    b, i = pl.program_id(0), pl.program_id(1)

    @pl.when(jnp.logical_and(b == 0, i == 0))
    def _cast():
        w1b_ref[...] = w1_ref[...].astype(jnp.bfloat16)
        w2b_ref[...] = w2_ref[...].astype(jnp.bfloat16)

    srow = _ln_rows(state_ref[0], og_ref[...], obn_ref[...])
    orow = _dot(srow, ow_ref[...].T) + ob_ref[...]
    x = res_ref[0] + orow
    u = _ln_rows(x, l2g_ref[...], l2b_ref[...])
    m1 = _gelu(_dot(u.astype(jnp.bfloat16), w1b_ref[...].T) + b1_ref[...])
    out_ref[0] = x + _dot(m1.astype(jnp.bfloat16), w2b_ref[...].T) + b2_ref[...]


def kernel(x_sacc, x_full, params):
    p = params
    B, N, _ = x_sacc.shape
    nb = N // BLK
    r1 = lambda v: v.reshape(1, -1)

    xf_blocks = x_full.reshape(B * nb, BLK, BD)
    xmid, stdv, maxv = pl.pallas_call(
        _periph_stage1_kernel,
        grid=(B * nb,),
        in_specs=[
            pl.BlockSpec((1, BLK, BD), lambda i: (i, 0, 0)),
            pl.BlockSpec((256, BD), lambda i: (0, 0)),
            pl.BlockSpec((256, 1), lambda i: (0, 0)),
        ],
        out_specs=[
            pl.BlockSpec((1, 256, BLK), lambda i: (i, 0, 0)),
            pl.BlockSpec((1, 1, BD), lambda i: (i, 0, 0)),
            pl.BlockSpec((1, 1, BD), lambda i: (i, 0, 0)),
        ],
        out_shape=[
            jax.ShapeDtypeStruct((B * nb, 256, BLK), jnp.float32),
            jax.ShapeDtypeStruct((B * nb, 1, BD), jnp.float32),
            jax.ShapeDtypeStruct((B * nb, 1, BD), jnp.float32),
        ],
    )(xf_blocks, p['p_conv_proj_w'], p['p_conv_proj_b'].reshape(256, 1))

    conv_in = xmid.reshape(B * nb, 256 * BLK)
    wflat = p['p_conv_w'].reshape(256, 256 * BLK)
    pos = jnp.tile(p['p_pos'][:nb], (B, 1))
    fw, fb = p['f_in_w'], p['f_in_b']

    ctrl_out_shape = [
        jax.ShapeDtypeStruct((B, nb), jnp.float32),
        jax.ShapeDtypeStruct((B, TOPK), jnp.int32),
        jax.ShapeDtypeStruct((B, TOPK), jnp.float32),
        jax.ShapeDtypeStruct((B, D), jnp.float32),
        jax.ShapeDtypeStruct((B, D), jnp.float32),
        jax.ShapeDtypeStruct((B, D), jnp.float32),
    ]
    (pmap_flat, state, scores, ti, tw, qf, ks, vs) = pl.pallas_call(
        _periph_ctrl_kernel,
        out_shape=[
            jax.ShapeDtypeStruct((B * nb, D), jnp.float32),
            jax.ShapeDtypeStruct((B, D), jnp.float32),
        ] + ctrl_out_shape,
    )(conv_in, wflat, r1(p['p_conv_b']), stdv.reshape(B * nb, BD),
      p['p_std_w'], r1(p['p_std_b']), maxv.reshape(B * nb, BD),
      p['p_max_w'], r1(p['p_max_b']), p['p_proj_w'], r1(p['p_proj_b']),
      r1(p['p_norm_g']), r1(p['p_norm_b']), pos,
      p['c_q_w'], r1(p['c_q_b']), p['c_k_w'], r1(p['c_k_b']),
      r1(p['f_n1_g']), r1(p['f_n1_b']), fw, r1(fb))
    pmap = pmap_flat.reshape(B, nb, D)

    bm = 512
    x_rows = x_sacc.reshape(B * N, D)
    ktab, vtab = pl.pallas_call(
        _kv_kernel,
        grid=(B * N // bm,),
        in_specs=[pl.BlockSpec((bm, D), lambda i: (i, 0))] +
                 [pl.BlockSpec((1, D), lambda i: (0, 0))] * 4 +
                 [pl.BlockSpec((3 * D, D), lambda i: (0, 0)),
                  pl.BlockSpec((1, 3 * D), lambda i: (0, 0))],
        out_specs=[pl.BlockSpec((bm, D), lambda i: (i, 0))] * 2,
        out_shape=[jax.ShapeDtypeStruct((B * N, D), jnp.float32)] * 2,
    )(x_rows, r1(p['ln1_g']), r1(p['ln1_b']), r1(p['f_n1_g']),
      r1(p['f_n1_b']), fw, r1(fb))
    kt3 = ktab.reshape(B, N, D)
    vt3 = vtab.reshape(B, N, D)

    head_mask = (lax.broadcasted_iota(jnp.int32, (D, H), 0) // DH ==
                 lax.broadcasted_iota(jnp.int32, (D, H), 1)).astype(jnp.float32)
    expander = head_mask.T

    controller = pl.pallas_call(_controller_kernel, out_shape=ctrl_out_shape)

    fps, flogits = [], []
    acc_starts = []
    for t in range(NSACC):
        if t > 0:
            scores, ti, tw, qf, ks, vs = controller(
                state, pmap, p['c_q_w'], r1(p['c_q_b']), p['c_k_w'],
                r1(p['c_k_b']), r1(p['f_n1_g']), r1(p['f_n1_b']), fw, r1(fb))
        fps.append(ti[:, 0] * BLK)
        flogits.append(scores)
        starts = jnp.clip(ti * BLK - WS // 2, 0, N - WS) // 8

        qmat = qf[:, :, None] * head_mask[None]
        astack = (jnp.stack(acc_starts, axis=1) if acc_starts
                  else jnp.zeros((B, 1), jnp.int32))
        nacc = len(acc_starts)

        ctxv = pl.pallas_call(
            functools.partial(_foveal_attn_kernel, nacc=nacc),
            grid=(B,),
            in_specs=[pl.BlockSpec(memory_space=pltpu.SMEM),
                      pl.BlockSpec(memory_space=pltpu.SMEM),
                      pl.BlockSpec((1, N, D), lambda b: (b, 0, 0)),
                      pl.BlockSpec((1, N, D), lambda b: (b, 0, 0)),
                      pl.BlockSpec((1, D, H), lambda b: (b, 0, 0)),
                      pl.BlockSpec((1, 1, D), lambda b: (b, 0, 0)),
                      pl.BlockSpec((1, 1, D), lambda b: (b, 0, 0)),
                      pl.BlockSpec((H, D), lambda b: (0, 0))],
            out_specs=pl.BlockSpec((1, TOPK, D), lambda b: (b, 0, 0)),
            out_shape=jax.ShapeDtypeStruct((B, TOPK, D), jnp.float32),
        )(starts, astack, kt3, vt3, qmat, ks.reshape(B, 1, D),
          vs.reshape(B, 1, D), expander)
        ctxv32 = ctxv.reshape(B * TOPK, D)

        acc_starts.append(starts[:, 0])
        astack2 = jnp.stack(acc_starts, axis=1)

        tt = jnp.array([[t / NSACC]], dtype=jnp.float32)
        a1 = _gelu(tt @ p['g1_w'].T + p['g1_b'])
        alpha = jax.nn.sigmoid(a1 @ p['g2_w'].T + p['g2_b'])

        st32 = jnp.repeat(state, TOPK, axis=0)
        state, pmap = pl.pallas_call(
            functools.partial(_epilogue_kernel, nacc=t + 1),
            in_specs=[pl.BlockSpec(memory_space=pltpu.SMEM),
                      pl.BlockSpec(memory_space=pltpu.SMEM)] +
                     [pl.BlockSpec()] * 20 +
                     [pl.BlockSpec(memory_space=pl.ANY)],
            out_shape=[jax.ShapeDtypeStruct((B, D), jnp.float32),
                       jax.ShapeDtypeStruct((B, nb, D), jnp.float32)],
            scratch_shapes=[pltpu.VMEM(((t + 1) * WS, D), jnp.float32),
                            pltpu.SemaphoreType.DMA],
        )(astack2, alpha, ctxv32, st32, tw, p['f_out_w'], r1(p['f_out_b']),
          r1(p['f_n2_g']), r1(p['f_n2_b']), p['f_ffn1_w'], r1(p['f_ffn1_b']),
          p['f_ffn2_w'], r1(p['f_ffn2_b']), pmap, r1(p['m_norm_g']),
          r1(p['m_norm_b']), p['m_in_w'], r1(p['m_in_b']), p['m_out_w'],
          r1(p['m_out_b']), r1(p['ln1_g']), r1(p['ln1_b']), x_sacc)

    bm2 = 128
    out = pl.pallas_call(
        _final_kernel,
        grid=(B, N // bm2),
        in_specs=[
            pl.BlockSpec((1, bm2, D), lambda b, i: (b, i, 0)),
            pl.BlockSpec((1, 1, D), lambda b, i: (b, 0, 0)),
        ] + [pl.BlockSpec((1, D), lambda b, i: (0, 0))] * 2 + [
            pl.BlockSpec((D, D), lambda b, i: (0, 0)),
            pl.BlockSpec((1, D), lambda b, i: (0, 0)),
            pl.BlockSpec((1, D), lambda b, i: (0, 0)),
            pl.BlockSpec((1, D), lambda b, i: (0, 0)),
            pl.BlockSpec((4 * D, D), lambda b, i: (0, 0)),
            pl.BlockSpec((1, 4 * D), lambda b, i: (0, 0)),
            pl.BlockSpec((D, 4 * D), lambda b, i: (0, 0)),
            pl.BlockSpec((1, D), lambda b, i: (0, 0)),
        ],
        out_specs=pl.BlockSpec((1, bm2, D), lambda b, i: (b, i, 0)),
        out_shape=jax.ShapeDtypeStruct((B, N, D), jnp.float32),
        scratch_shapes=[pltpu.VMEM((4 * D, D), jnp.bfloat16),
                        pltpu.VMEM((D, 4 * D), jnp.bfloat16)],
    )(x_sacc, state.reshape(B, 1, D), r1(p['o_norm_g']), r1(p['o_norm_b']),
      p['o_w'], r1(p['o_b']), r1(p['ln2_g']), r1(p['ln2_b']),
      p['mlp1_w'], r1(p['mlp1_b']), p['mlp2_w'], r1(p['mlp2_b']))

    return out, jnp.stack(fps), jnp.stack(flogits)


# speedup vs baseline: 1.2598x; 1.2598x over previous
import functools
import math

import jax
import jax.numpy as jnp
from jax import lax
from jax.experimental import pallas as pl
from jax.experimental.pallas import tpu as pltpu

D = 1024
BD = 2048
H = 16
DH = 64
BLK = 64
WS = 128
NSACC = 2
TOPK = 16


def _dot(a, b):
    return jnp.dot(a, b, preferred_element_type=jnp.float32)


def _ln_rows(x, g, b, eps=1e-5):
    m = jnp.mean(x, axis=-1, keepdims=True)
    v = jnp.mean((x - m) ** 2, axis=-1, keepdims=True)
    return (x - m) * lax.rsqrt(v + eps) * g + b


def _gelu(x):
    return 0.5 * x * (1.0 + lax.erf(x * (2.0 ** -0.5)))


def _ctrl_compute(state, pm2, cqw, cqb, ckw, ckb, n1g, n1b, fw, fb):
    B = state.shape[0]
    nb = pm2.shape[0] // B
    q = _dot(state, cqw.T) + cqb
    kk = _dot(pm2, ckw.T) + ckb
    sfull = _dot(q, kk.T) / math.sqrt(D)
    rows = [sfull[bi:bi + 1, bi * nb:(bi + 1) * nb] for bi in range(B)]
    scores = jnp.concatenate(rows, axis=0)

    iota = lax.broadcasted_iota(jnp.int32, (B, nb), 1)
    work = scores
    tvs, tis = [], []
    for _ in range(TOPK):
        m = jnp.max(work, axis=1, keepdims=True)
        idx = jnp.min(jnp.where(work == m, iota, nb), axis=1, keepdims=True)
        tvs.append(m)
        tis.append(idx)
        work = jnp.where(iota == idx, -jnp.inf, work)
    tv = jnp.concatenate(tvs, axis=1)
    ti = jnp.concatenate(tis, axis=1)
    e = jnp.exp((tv - tv[:, 0:1]) / 5.0)
    tw = e / jnp.sum(e, axis=1, keepdims=True)

    g = _ln_rows(state, n1g, n1b)
    qf = _dot(g, fw[:D, :].T) + fb[0:1, :D]
    ks = _dot(g, fw[D:2 * D, :].T) + fb[0:1, D:2 * D]
    vs = _dot(g, fw[2 * D:, :].T) + fb[0:1, 2 * D:]
    return scores, ti, tw, qf, ks, vs


def _periph_stage1_kernel(x_ref, wc_ref, bc_ref, xmid_ref, std_ref, max_ref):
    x = x_ref[0]
    xmid_ref[0] = lax.dot_general(
        wc_ref[...], x, (((1,), (1,)), ((), ())),
        preferred_element_type=jnp.float32) + bc_ref[...]
    mean = jnp.mean(x, axis=0, keepdims=True)
    var = jnp.sum((x - mean) ** 2, axis=0, keepdims=True) / (BLK - 1)
    std_ref[0] = jnp.sqrt(var)
    max_ref[0] = jnp.max(x, axis=0, keepdims=True)


def _periph_ctrl_kernel(cin_ref, wf_ref, cb_ref, sv_ref, sw_ref, sb_ref,
                        mv_ref, mw_ref, mb_ref, pp_ref, pb_ref, g_ref, b_ref,
                        pos_ref, cqw_ref, cqb_ref, ckw_ref, ckb_ref, n1g_ref,
                        n1b_ref, fw_ref, fb_ref,
                        pmap_ref, state_ref, scores_ref, ti_ref, tw_ref,
                        qf_ref, ks_ref, vs_ref):
    co = _dot(cin_ref[...], wf_ref[...].T) + cb_ref[...]
    so = _dot(sv_ref[...], sw_ref[...].T) + sb_ref[...]
    mo = _dot(mv_ref[...], mw_ref[...].T) + mb_ref[...]
    combined = jnp.concatenate([co, so, mo], axis=1)
    pre = _dot(combined, pp_ref[...].T) + pb_ref[...]
    pmap = _ln_rows(pre, g_ref[...], b_ref[...]) + pos_ref[...]
    pmap_ref[...] = pmap
    B = state_ref.shape[0]
    nb = pmap.shape[0] // B
    srows = [jnp.mean(pmap[bi * nb:(bi + 1) * nb], axis=0, keepdims=True)
             for bi in range(B)]
    state = jnp.concatenate(srows, axis=0)
    state_ref[...] = state
    scores, ti, tw, qf, ks, vs = _ctrl_compute(
        state, pmap, cqw_ref[...], cqb_ref[...], ckw_ref[...], ckb_ref[...],
        n1g_ref[...], n1b_ref[...], fw_ref[...], fb_ref[...])
    scores_ref[...] = scores
    ti_ref[...] = ti
    tw_ref[...] = tw
    qf_ref[...] = qf
    ks_ref[...] = ks
    vs_ref[...] = vs


def _kv_kernel(x_ref, l1g_ref, l1b_ref, n1g_ref, n1b_ref, fw_ref, fb_ref,
               kt_ref, vt_ref):
    h = _ln_rows(x_ref[...], l1g_ref[...], l1b_ref[...])
    g = _ln_rows(h, n1g_ref[...], n1b_ref[...])
    kt_ref[...] = _dot(g, fw_ref[D:2 * D, :].T) + fb_ref[0:1, D:2 * D]
    vt_ref[...] = _dot(g, fw_ref[2 * D:, :].T) + fb_ref[0:1, 2 * D:]


def _controller_kernel(state_ref, pmap_ref, cqw_ref, cqb_ref, ckw_ref,
                       ckb_ref, n1g_ref, n1b_ref, fw_ref, fb_ref,
                       scores_ref, ti_ref, tw_ref, qf_ref, ks_ref, vs_ref):
    B = state_ref.shape[0]
    nb = pmap_ref.shape[1]
    pm2 = pmap_ref[...].reshape(B * nb, D)
    scores, ti, tw, qf, ks, vs = _ctrl_compute(
        state_ref[...], pm2, cqw_ref[...], cqb_ref[...], ckw_ref[...],
        ckb_ref[...], n1g_ref[...], n1b_ref[...], fw_ref[...], fb_ref[...])
    scores_ref[...] = scores
    ti_ref[...] = ti
    tw_ref[...] = tw
    qf_ref[...] = qf
    ks_ref[...] = ks
    vs_ref[...] = vs


def _foveal_attn_kernel(starts_ref, astarts_ref, kt_ref, vt_ref, qm_ref,
                        ks_ref, vs_ref, ex_ref, ctxv_ref, *, nacc):
    b = pl.program_id(0)
    qm = qm_ref[0]
    ex = ex_ref[...]
    s_state = _dot(ks_ref[0], qm) / 8.0
    vs = vs_ref[0]

    saccs, vaccs = [], []
    for j in range(nacc):
        a0 = astarts_ref[b, j] * 8
        kacc = kt_ref[0, pl.ds(a0, WS), :]
        vacc = vt_ref[0, pl.ds(a0, WS), :]
        saccs.append(_dot(kacc, qm) / 8.0)
        vaccs.append(vacc)

    for k in range(TOPK):
        st = starts_ref[b, k] * 8
        kwin = kt_ref[0, pl.ds(st, WS), :]
        vwin = vt_ref[0, pl.ds(st, WS), :]
        sw = _dot(kwin, qm) / 8.0
        m = jnp.maximum(jnp.max(sw, axis=0, keepdims=True), s_state)
        for sa in saccs:
            m = jnp.maximum(m, jnp.max(sa, axis=0, keepdims=True))
        ew = jnp.exp(sw - m)
        es = jnp.exp(s_state - m)
        denom = jnp.sum(ew, axis=0, keepdims=True) + es
        eas = []
        for sa in saccs:
            ea = jnp.exp(sa - m)
            eas.append(ea)
            denom = denom + jnp.sum(ea, axis=0, keepdims=True)
        inv = 1.0 / denom
        ctxv = jnp.sum(vwin * _dot(ew * inv, ex), axis=0, keepdims=True)
        ctxv = ctxv + vs * _dot(es * inv, ex)
        for ea, vacc in zip(eas, vaccs):
            ctxv = ctxv + jnp.sum(vacc * _dot(ea * inv, ex), axis=0,
                                  keepdims=True)
        ctxv_ref[0, k:k + 1, :] = ctxv


def _epilogue_kernel(astarts_ref, alpha_ref, ctxv_ref, st32_ref, tw_ref,
                     ow_ref, ob_ref, n2g_ref, n2b_ref, w1_ref, b1_ref,
                     w2_ref, b2_ref, pmap_ref, ng_ref, nbg_ref, mw_ref,
                     mb_ref, wo_ref, bo_ref, l1g_ref, l1b_ref, x_hbm,
                     wst_ref, out_ref, accx_ref, sem, *, nacc):
    B, K = tw_ref.shape
    s = st32_ref[...] + _dot(ctxv_ref[...], ow_ref[...].T) + ob_ref[...]
    u = _ln_rows(s, n2g_ref[...], n2b_ref[...])
    m1 = _gelu(_dot(u, w1_ref[...].T) + b1_ref[...])
    s2 = s + _dot(m1, w2_ref[...].T) + b2_ref[...]
    for bi in range(B):
        wst_ref[bi:bi + 1, :] = _dot(tw_ref[bi:bi + 1, :],
                                     s2[bi * K:(bi + 1) * K, :])

    alpha = alpha_ref[0, 0]
    for bi in range(B):
        for j in range(nacc):
            a0 = astarts_ref[bi, j] * 8
            cp = pltpu.make_async_copy(
                x_hbm.at[bi, pl.ds(a0, WS), :],
                accx_ref.at[pl.ds(j * WS, WS), :], sem)
            cp.start()
            cp.wait()
        a = _ln_rows(accx_ref[...], l1g_ref[...], l1b_ref[...])
        pm = pmap_ref[bi]
        pn = _ln_rows(pm, ng_ref[...], nbg_ref[...])
        q = _dot(pn, mw_ref[:D, :].T) + mb_ref[0:1, :D]
        ka = _dot(a, mw_ref[D:2 * D, :].T) + mb_ref[0:1, D:2 * D]
        va = _dot(a, mw_ref[2 * D:, :].T) + mb_ref[0:1, 2 * D:]
        pieces = []
        for hh in range(H):
            sl = slice(hh * DH, (hh + 1) * DH)
            sc = _dot(q[:, sl], ka[:, sl].T) / 8.0
            sc = sc - jnp.max(sc, axis=1, keepdims=True)
            pr = jnp.exp(sc)
            pr = pr / jnp.sum(pr, axis=1, keepdims=True)
            pieces.append(_dot(pr, va[:, sl]))
        ctx = jnp.concatenate(pieces, axis=1)
        delta = _dot(ctx, wo_ref[...].T) + bo_ref[...]
        out_ref[bi] = pm + alpha * delta


def _final_kernel(res_ref, state_ref, og_ref, obn_ref, ow_ref, ob_ref,
                  l2g_ref, l2b_ref, w1_ref, b1_ref, w2_ref, b2_ref, out_ref):
    srow = _ln_rows(state_ref[0], og_ref[...], obn_ref[...])
    orow = _dot(srow, ow_ref[...].T) + ob_ref[...]
    x = res_ref[0] + orow
    u = _ln_rows(x, l2g_ref[...], l2b_ref[...])
    m1 = _gelu(_dot(u.astype(jnp.bfloat16), w1_ref[...].T) + b1_ref[...])
    out_ref[0] = x + _dot(m1.astype(jnp.bfloat16), w2_ref[...].T) + b2_ref[...]


def kernel(x_sacc, x_full, params):
    p = params
    B, N, _ = x_sacc.shape
    nb = N // BLK
    r1 = lambda v: v.reshape(1, -1)

    xf_blocks = x_full.reshape(B * nb, BLK, BD)
    xmid, stdv, maxv = pl.pallas_call(
        _periph_stage1_kernel,
        grid=(B * nb,),
        in_specs=[
            pl.BlockSpec((1, BLK, BD), lambda i: (i, 0, 0)),
            pl.BlockSpec((256, BD), lambda i: (0, 0)),
            pl.BlockSpec((256, 1), lambda i: (0, 0)),
        ],
        out_specs=[
            pl.BlockSpec((1, 256, BLK), lambda i: (i, 0, 0)),
            pl.BlockSpec((1, 1, BD), lambda i: (i, 0, 0)),
            pl.BlockSpec((1, 1, BD), lambda i: (i, 0, 0)),
        ],
        out_shape=[
            jax.ShapeDtypeStruct((B * nb, 256, BLK), jnp.float32),
            jax.ShapeDtypeStruct((B * nb, 1, BD), jnp.float32),
            jax.ShapeDtypeStruct((B * nb, 1, BD), jnp.float32),
        ],
    )(xf_blocks, p['p_conv_proj_w'], p['p_conv_proj_b'].reshape(256, 1))

    conv_in = xmid.reshape(B * nb, 256 * BLK)
    wflat = p['p_conv_w'].reshape(256, 256 * BLK)
    pos = jnp.tile(p['p_pos'][:nb], (B, 1))
    fw, fb = p['f_in_w'], p['f_in_b']

    ctrl_out_shape = [
        jax.ShapeDtypeStruct((B, nb), jnp.float32),
        jax.ShapeDtypeStruct((B, TOPK), jnp.int32),
        jax.ShapeDtypeStruct((B, TOPK), jnp.float32),
        jax.ShapeDtypeStruct((B, D), jnp.float32),
        jax.ShapeDtypeStruct((B, D), jnp.float32),
        jax.ShapeDtypeStruct((B, D), jnp.float32),
    ]
    (pmap_flat, state, scores, ti, tw, qf, ks, vs) = pl.pallas_call(
        _periph_ctrl_kernel,
        out_shape=[
            jax.ShapeDtypeStruct((B * nb, D), jnp.float32),
            jax.ShapeDtypeStruct((B, D), jnp.float32),
        ] + ctrl_out_shape,
    )(conv_in, wflat, r1(p['p_conv_b']), stdv.reshape(B * nb, BD),
      p['p_std_w'], r1(p['p_std_b']), maxv.reshape(B * nb, BD),
      p['p_max_w'], r1(p['p_max_b']), p['p_proj_w'], r1(p['p_proj_b']),
      r1(p['p_norm_g']), r1(p['p_norm_b']), pos,
      p['c_q_w'], r1(p['c_q_b']), p['c_k_w'], r1(p['c_k_b']),
      r1(p['f_n1_g']), r1(p['f_n1_b']), fw, r1(fb))
    pmap = pmap_flat.reshape(B, nb, D)

    bm = 512
    x_rows = x_sacc.reshape(B * N, D)
    ktab, vtab = pl.pallas_call(
        _kv_kernel,
        grid=(B * N // bm,),
        in_specs=[pl.BlockSpec((bm, D), lambda i: (i, 0))] +
                 [pl.BlockSpec((1, D), lambda i: (0, 0))] * 4 +
                 [pl.BlockSpec((3 * D, D), lambda i: (0, 0)),
                  pl.BlockSpec((1, 3 * D), lambda i: (0, 0))],
        out_specs=[pl.BlockSpec((bm, D), lambda i: (i, 0))] * 2,
        out_shape=[jax.ShapeDtypeStruct((B * N, D), jnp.float32)] * 2,
    )(x_rows, r1(p['ln1_g']), r1(p['ln1_b']), r1(p['f_n1_g']),
      r1(p['f_n1_b']), fw, r1(fb))
    kt3 = ktab.reshape(B, N, D)
    vt3 = vtab.reshape(B, N, D)

    head_mask = (lax.broadcasted_iota(jnp.int32, (D, H), 0) // DH ==
                 lax.broadcasted_iota(jnp.int32, (D, H), 1)).astype(jnp.float32)
    expander = head_mask.T

    controller = pl.pallas_call(_controller_kernel, out_shape=ctrl_out_shape)

    fps, flogits = [], []
    acc_starts = []
    for t in range(NSACC):
        if t > 0:
            scores, ti, tw, qf, ks, vs = controller(
                state, pmap, p['c_q_w'], r1(p['c_q_b']), p['c_k_w'],
                r1(p['c_k_b']), r1(p['f_n1_g']), r1(p['f_n1_b']), fw, r1(fb))
        fps.append(ti[:, 0] * BLK)
        flogits.append(scores)
        starts = jnp.clip(ti * BLK - WS // 2, 0, N - WS) // 8

        qmat = qf[:, :, None] * head_mask[None]
        astack = (jnp.stack(acc_starts, axis=1) if acc_starts
                  else jnp.zeros((B, 1), jnp.int32))
        nacc = len(acc_starts)

        ctxv = pl.pallas_call(
            functools.partial(_foveal_attn_kernel, nacc=nacc),
            grid=(B,),
            in_specs=[pl.BlockSpec(memory_space=pltpu.SMEM),
                      pl.BlockSpec(memory_space=pltpu.SMEM),
                      pl.BlockSpec((1, N, D), lambda b: (b, 0, 0)),
                      pl.BlockSpec((1, N, D), lambda b: (b, 0, 0)),
                      pl.BlockSpec((1, D, H), lambda b: (b, 0, 0)),
                      pl.BlockSpec((1, 1, D), lambda b: (b, 0, 0)),
                      pl.BlockSpec((1, 1, D), lambda b: (b, 0, 0)),
                      pl.BlockSpec((H, D), lambda b: (0, 0))],
            out_specs=pl.BlockSpec((1, TOPK, D), lambda b: (b, 0, 0)),
            out_shape=jax.ShapeDtypeStruct((B, TOPK, D), jnp.float32),
        )(starts, astack, kt3, vt3, qmat, ks.reshape(B, 1, D),
          vs.reshape(B, 1, D), expander)
        ctxv32 = ctxv.reshape(B * TOPK, D)

        acc_starts.append(starts[:, 0])
        astack2 = jnp.stack(acc_starts, axis=1)

        tt = jnp.array([[t / NSACC]], dtype=jnp.float32)
        a1 = _gelu(tt @ p['g1_w'].T + p['g1_b'])
        alpha = jax.nn.sigmoid(a1 @ p['g2_w'].T + p['g2_b'])

        st32 = jnp.repeat(state, TOPK, axis=0)
        state, pmap = pl.pallas_call(
            functools.partial(_epilogue_kernel, nacc=t + 1),
            in_specs=[pl.BlockSpec(memory_space=pltpu.SMEM),
                      pl.BlockSpec(memory_space=pltpu.SMEM)] +
                     [pl.BlockSpec()] * 20 +
                     [pl.BlockSpec(memory_space=pl.ANY)],
            out_shape=[jax.ShapeDtypeStruct((B, D), jnp.float32),
                       jax.ShapeDtypeStruct((B, nb, D), jnp.float32)],
            scratch_shapes=[pltpu.VMEM(((t + 1) * WS, D), jnp.float32),
                            pltpu.SemaphoreType.DMA],
        )(astack2, alpha, ctxv32, st32, tw, p['f_out_w'], r1(p['f_out_b']),
          r1(p['f_n2_g']), r1(p['f_n2_b']), p['f_ffn1_w'], r1(p['f_ffn1_b']),
          p['f_ffn2_w'], r1(p['f_ffn2_b']), pmap, r1(p['m_norm_g']),
          r1(p['m_norm_b']), p['m_in_w'], r1(p['m_in_b']), p['m_out_w'],
          r1(p['m_out_b']), r1(p['ln1_g']), r1(p['ln1_b']), x_sacc)

    bm2 = 256
    out = pl.pallas_call(
        _final_kernel,
        grid=(B, N // bm2),
        in_specs=[
            pl.BlockSpec((1, bm2, D), lambda b, i: (b, i, 0)),
            pl.BlockSpec((1, 1, D), lambda b, i: (b, 0, 0)),
        ] + [pl.BlockSpec((1, D), lambda b, i: (0, 0))] * 2 + [
            pl.BlockSpec((D, D), lambda b, i: (0, 0)),
            pl.BlockSpec((1, D), lambda b, i: (0, 0)),
            pl.BlockSpec((1, D), lambda b, i: (0, 0)),
            pl.BlockSpec((1, D), lambda b, i: (0, 0)),
            pl.BlockSpec((4 * D, D), lambda b, i: (0, 0)),
            pl.BlockSpec((1, 4 * D), lambda b, i: (0, 0)),
            pl.BlockSpec((D, 4 * D), lambda b, i: (0, 0)),
            pl.BlockSpec((1, D), lambda b, i: (0, 0)),
        ],
        out_specs=pl.BlockSpec((1, bm2, D), lambda b, i: (b, i, 0)),
        out_shape=jax.ShapeDtypeStruct((B, N, D), jnp.float32),
    )(x_sacc, state.reshape(B, 1, D), r1(p['o_norm_g']), r1(p['o_norm_b']),
      p['o_w'], r1(p['o_b']), r1(p['ln2_g']), r1(p['ln2_b']),
      p['mlp1_w'].astype(jnp.bfloat16), r1(p['mlp1_b']),
      p['mlp2_w'].astype(jnp.bfloat16), r1(p['mlp2_b']))

    return out, jnp.stack(fps), jnp.stack(flogits)


# speedup vs baseline: 1.3202x; 1.0480x over previous
import functools
import math

import jax
import jax.numpy as jnp
from jax import lax
from jax.experimental import pallas as pl
from jax.experimental.pallas import tpu as pltpu

D = 1024
BD = 2048
H = 16
DH = 64
BLK = 64
WS = 128
NSACC = 2
TOPK = 16


def _dot(a, b):
    return jnp.dot(a, b, preferred_element_type=jnp.float32)


def _dotg(a, b, dims):
    return lax.dot_general(a, b, (dims, ((), ())),
                           preferred_element_type=jnp.float32)


def _ln_rows(x, g, b, eps=1e-5):
    m = jnp.mean(x, axis=-1, keepdims=True)
    v = jnp.mean((x - m) ** 2, axis=-1, keepdims=True)
    return (x - m) * lax.rsqrt(v + eps) * g + b


def _gelu(x):
    return 0.5 * x * (1.0 + lax.erf(x * (2.0 ** -0.5)))


def _ctrl_compute(state, pm2, cqw, cqb, ckw, ckb, n1g, n1b, fw, fb, bqc,
                  mask):
    B = state.shape[0]
    nb = pm2.shape[0] // B
    q = _dot(state, cqw.T) + cqb
    kk = _dot(pm2, ckw.T) + ckb
    sfull = _dot(q, kk.T) / math.sqrt(D)
    rows = [sfull[bi:bi + 1, bi * nb:(bi + 1) * nb] for bi in range(B)]
    scores = jnp.concatenate(rows, axis=0)

    iota = lax.broadcasted_iota(jnp.int32, (B, nb), 1)
    work = scores
    tvs, tis = [], []
    for _ in range(TOPK):
        m = jnp.max(work, axis=1, keepdims=True)
        idx = jnp.min(jnp.where(work == m, iota, nb), axis=1, keepdims=True)
        tvs.append(m)
        tis.append(idx)
        work = jnp.where(iota == idx, -jnp.inf, work)
    tv = jnp.concatenate(tvs, axis=1)
    ti = jnp.concatenate(tis, axis=1)
    e = jnp.exp((tv - tv[:, 0:1]) / 5.0)
    tw = e / jnp.sum(e, axis=1, keepdims=True)

    g = _ln_rows(state, n1g, n1b)
    wq, wk = fw[:D, :], fw[D:2 * D, :]
    bk, bv = fb[0:1, D:2 * D], fb[0:1, 2 * D:]
    qfT = _dotg(wq, g, ((1,), (1,))) + bqc
    ks = _dot(g, wk.T) + bk
    vstate = _dot(g, fw[2 * D:, :].T) + bv
    combs, sbias, sst = [], [], []
    for bi in range(B):
        qmat = mask * qfT[:, bi:bi + 1]
        combs.append(_dotg(wk, qmat, ((0,), (0,))) / 8.0)
        sbias.append(_dot(bk, qmat) / 8.0)
        sst.append(_dot(ks[bi:bi + 1, :], qmat) / 8.0)
    return scores, ti, tw, combs, sbias, sst, vstate


def _write_ctrl(refs, vals):
    (scores_ref, ti_ref, tw_ref, comb_ref, sb_ref, sst_ref, vs_ref) = refs
    scores, ti, tw, combs, sbias, sst, vstate = vals
    scores_ref[...] = scores
    ti_ref[...] = ti
    tw_ref[...] = tw
    for bi in range(len(combs)):
        comb_ref[bi] = combs[bi]
        sb_ref[bi] = sbias[bi]
        sst_ref[bi] = sst[bi]
    vs_ref[...] = vstate


def _periph_stage1_kernel(x_ref, wc_ref, bc_ref, xmid_ref, std_ref, max_ref):
    x = x_ref[0]
    xmid_ref[0] = _dotg(wc_ref[...], x, ((1,), (1,))) + bc_ref[...]
    mean = jnp.mean(x, axis=0, keepdims=True)
    var = jnp.sum((x - mean) ** 2, axis=0, keepdims=True) / (BLK - 1)
    std_ref[0] = jnp.sqrt(var)
    max_ref[0] = jnp.max(x, axis=0, keepdims=True)


def _periph_ctrl_kernel(cin_ref, wf_ref, cb_ref, sv_ref, sw_ref, sb_ref,
                        mv_ref, mw_ref, mb_ref, pp_ref, pb_ref, g_ref, b_ref,
                        pos_ref, cqw_ref, cqb_ref, ckw_ref, ckb_ref, n1g_ref,
                        n1b_ref, fw_ref, fb_ref, bqc_ref, mask_ref,
                        pmap_ref, state_ref, *ctrl_refs):
    co = _dot(cin_ref[...], wf_ref[...].T) + cb_ref[...]
    so = _dot(sv_ref[...], sw_ref[...].T) + sb_ref[...]
    mo = _dot(mv_ref[...], mw_ref[...].T) + mb_ref[...]
    combined = jnp.concatenate([co, so, mo], axis=1)
    pre = _dot(combined, pp_ref[...].T) + pb_ref[...]
    pmap = _ln_rows(pre, g_ref[...], b_ref[...]) + pos_ref[...]
    pmap_ref[...] = pmap
    B = state_ref.shape[0]
    nb = pmap.shape[0] // B
    srows = [jnp.mean(pmap[bi * nb:(bi + 1) * nb], axis=0, keepdims=True)
             for bi in range(B)]
    state = jnp.concatenate(srows, axis=0)
    state_ref[...] = state
    vals = _ctrl_compute(state, pmap, cqw_ref[...], cqb_ref[...], ckw_ref[...],
                         ckb_ref[...], n1g_ref[...], n1b_ref[...], fw_ref[...],
                         fb_ref[...], bqc_ref[...], mask_ref[...])
    _write_ctrl(ctrl_refs, vals)


def _controller_kernel(state_ref, pmap_ref, cqw_ref, cqb_ref, ckw_ref,
                       ckb_ref, n1g_ref, n1b_ref, fw_ref, fb_ref, bqc_ref,
                       mask_ref, *ctrl_refs):
    B = state_ref.shape[0]
    nb = pmap_ref.shape[1]
    pm2 = pmap_ref[...].reshape(B * nb, D)
    vals = _ctrl_compute(state_ref[...], pm2, cqw_ref[...], cqb_ref[...],
                         ckw_ref[...], ckb_ref[...], n1g_ref[...],
                         n1b_ref[...], fw_ref[...], fb_ref[...], bqc_ref[...],
                         mask_ref[...])
    _write_ctrl(ctrl_refs, vals)


def _kv_score_kernel(x_ref, l1g_ref, l1b_ref, n1g_ref, n1b_ref, wv_ref,
                     bv_ref, comb_ref, sb_ref, vt_ref, sall_ref):
    h = _ln_rows(x_ref[...], l1g_ref[...], l1b_ref[...])
    g = _ln_rows(h, n1g_ref[...], n1b_ref[...])
    vt_ref[...] = _dot(g, wv_ref[...].T) + bv_ref[...]
    sall_ref[...] = _dot(g, comb_ref[0]) + sb_ref[0]


def _score_kernel(x_ref, l1g_ref, l1b_ref, n1g_ref, n1b_ref, comb_ref,
                  sb_ref, sall_ref):
    h = _ln_rows(x_ref[...], l1g_ref[...], l1b_ref[...])
    g = _ln_rows(h, n1g_ref[...], n1b_ref[...])
    sall_ref[...] = _dot(g, comb_ref[0]) + sb_ref[0]


def _foveal_attn_kernel(starts_ref, astarts_ref, sall_ref, vt_ref, sst_ref,
                        vs_ref, ex_ref, ctxv_ref, *, nacc):
    b = pl.program_id(0)
    ex = ex_ref[...]
    s_state = sst_ref[0]
    vs = vs_ref[0]

    saccs, vaccs = [], []
    for j in range(nacc):
        a0 = astarts_ref[b, j] * 8
        saccs.append(sall_ref[0, pl.ds(a0, WS), :])
        vaccs.append(vt_ref[0, pl.ds(a0, WS), :])

    for k in range(TOPK):
        st = starts_ref[b, k] * 8
        sw = sall_ref[0, pl.ds(st, WS), :]
        vwin = vt_ref[0, pl.ds(st, WS), :]
        m = jnp.maximum(jnp.max(sw, axis=0, keepdims=True), s_state)
        for sa in saccs:
            m = jnp.maximum(m, jnp.max(sa, axis=0, keepdims=True))
        ew = jnp.exp(sw - m)
        es = jnp.exp(s_state - m)
        denom = jnp.sum(ew, axis=0, keepdims=True) + es
        eas = []
        for sa in saccs:
            ea = jnp.exp(sa - m)
            eas.append(ea)
            denom = denom + jnp.sum(ea, axis=0, keepdims=True)
        inv = 1.0 / denom
        ctxv = jnp.sum(vwin * _dot(ew * inv, ex), axis=0, keepdims=True)
        ctxv = ctxv + vs * _dot(es * inv, ex)
        for ea, vacc in zip(eas, vaccs):
            ctxv = ctxv + jnp.sum(vacc * _dot(ea * inv, ex), axis=0,
                                  keepdims=True)
        ctxv_ref[0, k:k + 1, :] = ctxv


def _epilogue_kernel(astarts_ref, alpha_ref, ctxv_ref, st32_ref, tw_ref,
                     ow_ref, ob_ref, n2g_ref, n2b_ref, w1_ref, b1_ref,
                     w2_ref, b2_ref, pmap_ref, ng_ref, nbg_ref, mw_ref,
                     mb_ref, wo_ref, bo_ref, l1g_ref, l1b_ref, x_hbm,
                     wst_ref, out_ref, accx_ref, sem, *, nacc):
    B, K = tw_ref.shape
    s = st32_ref[...] + _dot(ctxv_ref[...], ow_ref[...].T) + ob_ref[...]
    u = _ln_rows(s, n2g_ref[...], n2b_ref[...])
    m1 = _gelu(_dot(u, w1_ref[...].T) + b1_ref[...])
    s2 = s + _dot(m1, w2_ref[...].T) + b2_ref[...]
    for bi in range(B):
        wst_ref[bi:bi + 1, :] = _dot(tw_ref[bi:bi + 1, :],
                                     s2[bi * K:(bi + 1) * K, :])

    alpha = alpha_ref[0, 0]
    for bi in range(B):
        for j in range(nacc):
            a0 = astarts_ref[bi, j] * 8
            cp = pltpu.make_async_copy(
                x_hbm.at[bi, pl.ds(a0, WS), :],
                accx_ref.at[pl.ds(j * WS, WS), :], sem)
            cp.start()
            cp.wait()
        a = _ln_rows(accx_ref[...], l1g_ref[...], l1b_ref[...])
        pm = pmap_ref[bi]
        pn = _ln_rows(pm, ng_ref[...], nbg_ref[...])
        q = _dot(pn, mw_ref[:D, :].T) + mb_ref[0:1, :D]
        ka = _dot(a, mw_ref[D:2 * D, :].T) + mb_ref[0:1, D:2 * D]
        va = _dot(a, mw_ref[2 * D:, :].T) + mb_ref[0:1, 2 * D:]
        pieces = []
        for hh in range(H):
            sl = slice(hh * DH, (hh + 1) * DH)
            sc = _dot(q[:, sl], ka[:, sl].T) / 8.0
            sc = sc - jnp.max(sc, axis=1, keepdims=True)
            pr = jnp.exp(sc)
            pr = pr / jnp.sum(pr, axis=1, keepdims=True)
            pieces.append(_dot(pr, va[:, sl]))
        ctx = jnp.concatenate(pieces, axis=1)
        delta = _dot(ctx, wo_ref[...].T) + bo_ref[...]
        out_ref[bi] = pm + alpha * delta


def _final_kernel(res_ref, state_ref, og_ref, obn_ref, ow_ref, ob_ref,
                  l2g_ref, l2b_ref, w1_ref, b1_ref, w2_ref, b2_ref, out_ref):
    srow = _ln_rows(state_ref[0], og_ref[...], obn_ref[...])
    orow = _dot(srow, ow_ref[...].T) + ob_ref[...]
    x = res_ref[0] + orow
    u = _ln_rows(x, l2g_ref[...], l2b_ref[...])
    m1 = _gelu(_dot(u.astype(jnp.bfloat16), w1_ref[...].T) + b1_ref[...])
    out_ref[0] = x + _dot(m1.astype(jnp.bfloat16), w2_ref[...].T) + b2_ref[...]


def kernel(x_sacc, x_full, params):
    p = params
    B, N, _ = x_sacc.shape
    nb = N // BLK
    r1 = lambda v: v.reshape(1, -1)

    xf_blocks = x_full.reshape(B * nb, BLK, BD)
    xmid, stdv, maxv = pl.pallas_call(
        _periph_stage1_kernel,
        grid=(B * nb,),
        in_specs=[
            pl.BlockSpec((1, BLK, BD), lambda i: (i, 0, 0)),
            pl.BlockSpec((256, BD), lambda i: (0, 0)),
            pl.BlockSpec((256, 1), lambda i: (0, 0)),
        ],
        out_specs=[
            pl.BlockSpec((1, 256, BLK), lambda i: (i, 0, 0)),
            pl.BlockSpec((1, 1, BD), lambda i: (i, 0, 0)),
            pl.BlockSpec((1, 1, BD), lambda i: (i, 0, 0)),
        ],
        out_shape=[
            jax.ShapeDtypeStruct((B * nb, 256, BLK), jnp.float32),
            jax.ShapeDtypeStruct((B * nb, 1, BD), jnp.float32),
            jax.ShapeDtypeStruct((B * nb, 1, BD), jnp.float32),
        ],
    )(xf_blocks, p['p_conv_proj_w'], p['p_conv_proj_b'].reshape(256, 1))

    conv_in = xmid.reshape(B * nb, 256 * BLK)
    wflat = p['p_conv_w'].reshape(256, 256 * BLK)
    pos = jnp.tile(p['p_pos'][:nb], (B, 1))
    fw, fb = p['f_in_w'], p['f_in_b']
    bqc = fb[:D].reshape(D, 1)
    mask = (lax.broadcasted_iota(jnp.int32, (D, H), 0) // DH ==
            lax.broadcasted_iota(jnp.int32, (D, H), 1)).astype(jnp.float32)
    expander = mask.T

    ctrl_out_shape = [
        jax.ShapeDtypeStruct((B, nb), jnp.float32),
        jax.ShapeDtypeStruct((B, TOPK), jnp.int32),
        jax.ShapeDtypeStruct((B, TOPK), jnp.float32),
        jax.ShapeDtypeStruct((B, D, H), jnp.float32),
        jax.ShapeDtypeStruct((B, 1, H), jnp.float32),
        jax.ShapeDtypeStruct((B, 1, H), jnp.float32),
        jax.ShapeDtypeStruct((B, D), jnp.float32),
    ]

    (pmap_flat, state, scores, ti, tw, comb, sbias, sst, vs) = pl.pallas_call(
        _periph_ctrl_kernel,
        out_shape=[
            jax.ShapeDtypeStruct((B * nb, D), jnp.float32),
            jax.ShapeDtypeStruct((B, D), jnp.float32),
        ] + ctrl_out_shape,
    )(conv_in, wflat, r1(p['p_conv_b']), stdv.reshape(B * nb, BD),
      p['p_std_w'], r1(p['p_std_b']), maxv.reshape(B * nb, BD),
      p['p_max_w'], r1(p['p_max_b']), p['p_proj_w'], r1(p['p_proj_b']),
      r1(p['p_norm_g']), r1(p['p_norm_b']), pos,
      p['c_q_w'], r1(p['c_q_b']), p['c_k_w'], r1(p['c_k_b']),
      r1(p['f_n1_g']), r1(p['f_n1_b']), fw, r1(fb), bqc, mask)
    pmap = pmap_flat.reshape(B, nb, D)

    bm = 512
    ng = B * N // bm
    nbb = ng // B
    x_rows = x_sacc.reshape(B * N, D)
    vtab, sall = pl.pallas_call(
        _kv_score_kernel,
        grid=(ng,),
        in_specs=[pl.BlockSpec((bm, D), lambda i: (i, 0))] +
                 [pl.BlockSpec((1, D), lambda i: (0, 0))] * 4 +
                 [pl.BlockSpec((D, D), lambda i: (0, 0)),
                  pl.BlockSpec((1, D), lambda i: (0, 0)),
                  pl.BlockSpec((1, D, H), lambda i: (i // nbb, 0, 0)),
                  pl.BlockSpec((1, 1, H), lambda i: (i // nbb, 0, 0))],
        out_specs=[pl.BlockSpec((bm, D), lambda i: (i, 0)),
                   pl.BlockSpec((bm, H), lambda i: (i, 0))],
        out_shape=[jax.ShapeDtypeStruct((B * N, D), jnp.float32),
                   jax.ShapeDtypeStruct((B * N, H), jnp.float32)],
    )(x_rows, r1(p['ln1_g']), r1(p['ln1_b']), r1(p['f_n1_g']),
      r1(p['f_n1_b']), fw[2 * D:], r1(fb[2 * D:]), comb, sbias)
    vt3 = vtab.reshape(B, N, D)

    controller = pl.pallas_call(_controller_kernel, out_shape=ctrl_out_shape)
    score_call = pl.pallas_call(
        _score_kernel,
        grid=(ng,),
        in_specs=[pl.BlockSpec((bm, D), lambda i: (i, 0))] +
                 [pl.BlockSpec((1, D), lambda i: (0, 0))] * 4 +
                 [pl.BlockSpec((1, D, H), lambda i: (i // nbb, 0, 0)),
                  pl.BlockSpec((1, 1, H), lambda i: (i // nbb, 0, 0))],
        out_specs=pl.BlockSpec((bm, H), lambda i: (i, 0)),
        out_shape=jax.ShapeDtypeStruct((B * N, H), jnp.float32),
    )

    fps, flogits = [], []
    acc_starts = []
    for t in range(NSACC):
        if t > 0:
            scores, ti, tw, comb, sbias, sst, vs = controller(
                state, pmap, p['c_q_w'], r1(p['c_q_b']), p['c_k_w'],
                r1(p['c_k_b']), r1(p['f_n1_g']), r1(p['f_n1_b']), fw, r1(fb),
                bqc, mask)
            sall = score_call(x_rows, r1(p['ln1_g']), r1(p['ln1_b']),
                              r1(p['f_n1_g']), r1(p['f_n1_b']), comb, sbias)
        fps.append(ti[:, 0] * BLK)
        flogits.append(scores)
        starts = jnp.clip(ti * BLK - WS // 2, 0, N - WS) // 8
        sall3 = sall.reshape(B, N, H)

        astack = (jnp.stack(acc_starts, axis=1) if acc_starts
                  else jnp.zeros((B, 1), jnp.int32))
        nacc = len(acc_starts)

        ctxv = pl.pallas_call(
            functools.partial(_foveal_attn_kernel, nacc=nacc),
            grid=(B,),
            in_specs=[pl.BlockSpec(memory_space=pltpu.SMEM),
                      pl.BlockSpec(memory_space=pltpu.SMEM),
                      pl.BlockSpec((1, N, H), lambda b: (b, 0, 0)),
                      pl.BlockSpec((1, N, D), lambda b: (b, 0, 0)),
                      pl.BlockSpec((1, 1, H), lambda b: (b, 0, 0)),
                      pl.BlockSpec((1, 1, D), lambda b: (b, 0, 0)),
                      pl.BlockSpec((H, D), lambda b: (0, 0))],
            out_specs=pl.BlockSpec((1, TOPK, D), lambda b: (b, 0, 0)),
            out_shape=jax.ShapeDtypeStruct((B, TOPK, D), jnp.float32),
        )(starts, astack, sall3, vt3, sst, vs.reshape(B, 1, D), expander)
        ctxv32 = ctxv.reshape(B * TOPK, D)

        acc_starts.append(starts[:, 0])
        astack2 = jnp.stack(acc_starts, axis=1)

        tt = jnp.array([[t / NSACC]], dtype=jnp.float32)
        a1 = _gelu(tt @ p['g1_w'].T + p['g1_b'])
        alpha = jax.nn.sigmoid(a1 @ p['g2_w'].T + p['g2_b'])

        st32 = jnp.repeat(state, TOPK, axis=0)
        state, pmap = pl.pallas_call(
            functools.partial(_epilogue_kernel, nacc=t + 1),
            in_specs=[pl.BlockSpec(memory_space=pltpu.SMEM),
                      pl.BlockSpec(memory_space=pltpu.SMEM)] +
                     [pl.BlockSpec()] * 20 +
                     [pl.BlockSpec(memory_space=pl.ANY)],
            out_shape=[jax.ShapeDtypeStruct((B, D), jnp.float32),
                       jax.ShapeDtypeStruct((B, nb, D), jnp.float32)],
            scratch_shapes=[pltpu.VMEM(((t + 1) * WS, D), jnp.float32),
                            pltpu.SemaphoreType.DMA],
        )(astack2, alpha, ctxv32, st32, tw, p['f_out_w'], r1(p['f_out_b']),
          r1(p['f_n2_g']), r1(p['f_n2_b']), p['f_ffn1_w'], r1(p['f_ffn1_b']),
          p['f_ffn2_w'], r1(p['f_ffn2_b']), pmap, r1(p['m_norm_g']),
          r1(p['m_norm_b']), p['m_in_w'], r1(p['m_in_b']), p['m_out_w'],
          r1(p['m_out_b']), r1(p['ln1_g']), r1(p['ln1_b']), x_sacc)

    bm2 = 512
    out = pl.pallas_call(
        _final_kernel,
        grid=(B, N // bm2),
        in_specs=[
            pl.BlockSpec((1, bm2, D), lambda b, i: (b, i, 0)),
            pl.BlockSpec((1, 1, D), lambda b, i: (b, 0, 0)),
        ] + [pl.BlockSpec((1, D), lambda b, i: (0, 0))] * 2 + [
            pl.BlockSpec((D, D), lambda b, i: (0, 0)),
            pl.BlockSpec((1, D), lambda b, i: (0, 0)),
            pl.BlockSpec((1, D), lambda b, i: (0, 0)),
            pl.BlockSpec((1, D), lambda b, i: (0, 0)),
            pl.BlockSpec((4 * D, D), lambda b, i: (0, 0)),
            pl.BlockSpec((1, 4 * D), lambda b, i: (0, 0)),
            pl.BlockSpec((D, 4 * D), lambda b, i: (0, 0)),
            pl.BlockSpec((1, D), lambda b, i: (0, 0)),
        ],
        out_specs=pl.BlockSpec((1, bm2, D), lambda b, i: (b, i, 0)),
        out_shape=jax.ShapeDtypeStruct((B, N, D), jnp.float32),
    )(x_sacc, state.reshape(B, 1, D), r1(p['o_norm_g']), r1(p['o_norm_b']),
      p['o_w'], r1(p['o_b']), r1(p['ln2_g']), r1(p['ln2_b']),
      p['mlp1_w'].astype(jnp.bfloat16), r1(p['mlp1_b']),
      p['mlp2_w'].astype(jnp.bfloat16), r1(p['mlp2_b']))

    return out, jnp.stack(fps), jnp.stack(flogits)


# speedup vs baseline: 1.5520x; 1.1756x over previous
import functools
import math

import jax
import jax.numpy as jnp
from jax import lax
from jax.experimental import pallas as pl
from jax.experimental.pallas import tpu as pltpu

D = 1024
BD = 2048
H = 16
DH = 64
BLK = 64
WS = 128
NSACC = 2
TOPK = 16


def _dot(a, b):
    return jnp.dot(a, b, preferred_element_type=jnp.float32)


def _dotg(a, b, dims):
    return lax.dot_general(a, b, (dims, ((), ())),
                           preferred_element_type=jnp.float32)


def _ln_rows(x, g, b, eps=1e-5):
    m = jnp.mean(x, axis=-1, keepdims=True)
    v = jnp.mean((x - m) ** 2, axis=-1, keepdims=True)
    return (x - m) * lax.rsqrt(v + eps) * g + b


def _gelu(x):
    return 0.5 * x * (1.0 + lax.erf(x * (2.0 ** -0.5)))


def _ctrl_compute(state, pm2, cqw, cqb, ckw, ckb, n1g, n1b, fw, fb, bqc,
                  mask):
    B = state.shape[0]
    nb = pm2.shape[0] // B
    q = _dot(state, cqw.T) + cqb
    kk = _dot(pm2, ckw.T) + ckb
    sfull = _dot(q, kk.T) / math.sqrt(D)
    rows = [sfull[bi:bi + 1, bi * nb:(bi + 1) * nb] for bi in range(B)]
    scores = jnp.concatenate(rows, axis=0)

    iota = lax.broadcasted_iota(jnp.int32, (B, nb), 1)
    work = scores
    tvs, tis = [], []
    for _ in range(TOPK):
        m = jnp.max(work, axis=1, keepdims=True)
        idx = jnp.min(jnp.where(work == m, iota, nb), axis=1, keepdims=True)
        tvs.append(m)
        tis.append(idx)
        work = jnp.where(iota == idx, -jnp.inf, work)
    tv = jnp.concatenate(tvs, axis=1)
    ti = jnp.concatenate(tis, axis=1)
    e = jnp.exp((tv - tv[:, 0:1]) / 5.0)
    tw = e / jnp.sum(e, axis=1, keepdims=True)

    g = _ln_rows(state, n1g, n1b)
    wq, wk = fw[:D, :], fw[D:2 * D, :]
    bk, bv = fb[0:1, D:2 * D], fb[0:1, 2 * D:]
    qfT = _dotg(wq, g, ((1,), (1,))) + bqc
    ks = _dot(g, wk.T) + bk
    vstate = _dot(g, fw[2 * D:, :].T) + bv
    combs, sbias, sst = [], [], []
    for bi in range(B):
        qmat = mask * qfT[:, bi:bi + 1]
        combs.append(_dotg(wk, qmat, ((0,), (0,))) / 8.0)
        sbias.append(_dot(bk, qmat) / 8.0)
        sst.append(_dot(ks[bi:bi + 1, :], qmat) / 8.0)
    return scores, ti, tw, combs, sbias, sst, vstate


def _write_ctrl(refs, vals):
    (scores_ref, ti_ref, tw_ref, comb_ref, sb_ref, sst_ref, vs_ref) = refs
    scores, ti, tw, combs, sbias, sst, vstate = vals
    scores_ref[...] = scores
    ti_ref[...] = ti
    tw_ref[...] = tw
    for bi in range(len(combs)):
        comb_ref[bi] = combs[bi]
        sb_ref[bi] = sbias[bi]
        sst_ref[bi] = sst[bi]
    vs_ref[...] = vstate


def _periph_stage1_kernel(x_ref, wc_ref, bc_ref, xmid_ref, std_ref, max_ref,
                          *, gpb):
    x = x_ref[...]
    xmT = _dotg(wc_ref[...], x, ((1,), (1,))) + bc_ref[...]
    for g in range(gpb):
        xmid_ref[g] = xmT[:, g * BLK:(g + 1) * BLK]
        xs = x[g * BLK:(g + 1) * BLK]
        mean = jnp.mean(xs, axis=0, keepdims=True)
        var = jnp.sum((xs - mean) ** 2, axis=0, keepdims=True) / (BLK - 1)
        std_ref[g] = jnp.sqrt(var)
        max_ref[g] = jnp.max(xs, axis=0, keepdims=True)


def _periph_ctrl_kernel(cin_ref, wf_ref, cb_ref, sv_ref, sw_ref, sb_ref,
                        mv_ref, mw_ref, mb_ref, pp_ref, pb_ref, g_ref, b_ref,
                        pos_ref, cqw_ref, cqb_ref, ckw_ref, ckb_ref, n1g_ref,
                        n1b_ref, fw_ref, fb_ref, bqc_ref, mask_ref,
                        pmap_ref, state_ref, *ctrl_refs):
    co = _dot(cin_ref[...], wf_ref[...].T) + cb_ref[...]
    so = _dot(sv_ref[...], sw_ref[...].T) + sb_ref[...]
    mo = _dot(mv_ref[...], mw_ref[...].T) + mb_ref[...]
    combined = jnp.concatenate([co, so, mo], axis=1)
    pre = _dot(combined, pp_ref[...].T) + pb_ref[...]
    pmap = _ln_rows(pre, g_ref[...], b_ref[...]) + pos_ref[...]
    pmap_ref[...] = pmap
    B = state_ref.shape[0]
    nb = pmap.shape[0] // B
    srows = [jnp.mean(pmap[bi * nb:(bi + 1) * nb], axis=0, keepdims=True)
             for bi in range(B)]
    state = jnp.concatenate(srows, axis=0)
    state_ref[...] = state
    vals = _ctrl_compute(state, pmap, cqw_ref[...], cqb_ref[...], ckw_ref[...],
                         ckb_ref[...], n1g_ref[...], n1b_ref[...], fw_ref[...],
                         fb_ref[...], bqc_ref[...], mask_ref[...])
    _write_ctrl(ctrl_refs, vals)


def _controller_kernel(state_ref, pmap_ref, cqw_ref, cqb_ref, ckw_ref,
                       ckb_ref, n1g_ref, n1b_ref, fw_ref, fb_ref, bqc_ref,
                       mask_ref, *ctrl_refs):
    B = state_ref.shape[0]
    nb = pmap_ref.shape[1]
    pm2 = pmap_ref[...].reshape(B * nb, D)
    vals = _ctrl_compute(state_ref[...], pm2, cqw_ref[...], cqb_ref[...],
                         ckw_ref[...], ckb_ref[...], n1g_ref[...],
                         n1b_ref[...], fw_ref[...], fb_ref[...], bqc_ref[...],
                         mask_ref[...])
    _write_ctrl(ctrl_refs, vals)


def _kv_score_kernel(x_ref, l1g_ref, l1b_ref, n1g_ref, n1b_ref, wv_ref,
                     bv_ref, comb_ref, sb_ref, vt_ref, sall_ref):
    h = _ln_rows(x_ref[...], l1g_ref[...], l1b_ref[...])
    g = _ln_rows(h, n1g_ref[...], n1b_ref[...])
    vt_ref[...] = _dot(g, wv_ref[...].T) + bv_ref[...]
    sall_ref[...] = _dot(g, comb_ref[0]) + sb_ref[0]


def _score_kernel(x_ref, l1g_ref, l1b_ref, n1g_ref, n1b_ref, comb_ref,
                  sb_ref, sall_ref):
    h = _ln_rows(x_ref[...], l1g_ref[...], l1b_ref[...])
    g = _ln_rows(h, n1g_ref[...], n1b_ref[...])
    sall_ref[...] = _dot(g, comb_ref[0]) + sb_ref[0]


def _foveal_attn_kernel(starts_ref, astarts_ref, sall_ref, vt_ref, sst_ref,
                        vs_ref, ex_ref, ctxv_ref, *, nacc):
    b = pl.program_id(0)
    ex = ex_ref[...]
    s_state = sst_ref[0]
    vs = vs_ref[0]

    saccs, vaccs = [], []
    for j in range(nacc):
        a0 = astarts_ref[b, j] * 8
        saccs.append(sall_ref[0, pl.ds(a0, WS), :])
        vaccs.append(vt_ref[0, pl.ds(a0, WS), :])

    for k in range(TOPK):
        st = starts_ref[b, k] * 8
        sw = sall_ref[0, pl.ds(st, WS), :]
        vwin = vt_ref[0, pl.ds(st, WS), :]
        m = jnp.maximum(jnp.max(sw, axis=0, keepdims=True), s_state)
        for sa in saccs:
            m = jnp.maximum(m, jnp.max(sa, axis=0, keepdims=True))
        ew = jnp.exp(sw - m)
        es = jnp.exp(s_state - m)
        denom = jnp.sum(ew, axis=0, keepdims=True) + es
        eas = []
        for sa in saccs:
            ea = jnp.exp(sa - m)
            eas.append(ea)
            denom = denom + jnp.sum(ea, axis=0, keepdims=True)
        inv = 1.0 / denom
        ctxv = jnp.sum(vwin * _dot(ew * inv, ex), axis=0, keepdims=True)
        ctxv = ctxv + vs * _dot(es * inv, ex)
        for ea, vacc in zip(eas, vaccs):
            ctxv = ctxv + jnp.sum(vacc * _dot(ea * inv, ex), axis=0,
                                  keepdims=True)
        ctxv_ref[0, k:k + 1, :] = ctxv


def _dense_kernel(ctxv_ref, st32_ref, tw_ref, ow_ref, ob_ref, n2g_ref,
                  n2b_ref, w1_ref, b1_ref, w2_ref, b2_ref, wst_ref, acc_ref):
    j = pl.program_id(0)
    nj = pl.num_programs(0)
    B, K = tw_ref.shape
    s = st32_ref[...] + _dot(ctxv_ref[...], ow_ref[...].T) + ob_ref[...]
    u = _ln_rows(s, n2g_ref[...], n2b_ref[...])
    m1 = _gelu(_dot(u, w1_ref[...].T) + b1_ref[...])
    pj = _dot(m1, w2_ref[...].T)

    @pl.when(j == 0)
    def _init():
        acc_ref[...] = pj

    @pl.when(j > 0)
    def _acc():
        acc_ref[...] = acc_ref[...] + pj

    @pl.when(j == nj - 1)
    def _fin():
        s2 = s + acc_ref[...] + b2_ref[...]
        for bi in range(B):
            wst_ref[bi:bi + 1, :] = _dot(tw_ref[bi:bi + 1, :],
                                         s2[bi * K:(bi + 1) * K, :])


def _mattn_kernel(astarts_ref, alpha_ref, pmap_ref, ng_ref, nbg_ref, mw_ref,
                  mb_ref, wo_ref, bo_ref, l1g_ref, l1b_ref, x_hbm,
                  out_ref, qs_ref, ka_ref, va_ref, accx_ref, sem, *, nacc):
    j = pl.program_id(0)
    B = pmap_ref.shape[0]

    @pl.when(j == 0)
    def _dma():
        for bi in range(B):
            for jj in range(nacc):
                a0 = astarts_ref[bi, jj] * 8
                cp = pltpu.make_async_copy(
                    x_hbm.at[bi, pl.ds(a0, WS), :],
                    accx_ref.at[bi, pl.ds(jj * WS, WS), :], sem)
                cp.start()
                cp.wait()

    alpha = alpha_ref[0, 0]

    @pl.when(j == 0)
    def _q():
        for bi in range(B):
            pn = _ln_rows(pmap_ref[bi], ng_ref[...], nbg_ref[...])
            qs_ref[bi] = _dot(pn, mw_ref[...].T) + mb_ref[...]

    @pl.when(j == 1)
    def _k():
        for bi in range(B):
            a = _ln_rows(accx_ref[bi], l1g_ref[...], l1b_ref[...])
            ka_ref[bi] = _dot(a, mw_ref[...].T) + mb_ref[...]

    @pl.when(j == 2)
    def _v():
        for bi in range(B):
            a = _ln_rows(accx_ref[bi], l1g_ref[...], l1b_ref[...])
            va_ref[bi] = _dot(a, mw_ref[...].T) + mb_ref[...]
        for bi in range(B):
            q = qs_ref[bi]
            ka = ka_ref[bi]
            va = va_ref[bi]
            pieces = []
            for hh in range(H):
                sl = slice(hh * DH, (hh + 1) * DH)
                sc = _dot(q[:, sl], ka[:, sl].T) / 8.0
                sc = sc - jnp.max(sc, axis=1, keepdims=True)
                pr = jnp.exp(sc)
                pr = pr / jnp.sum(pr, axis=1, keepdims=True)
                pieces.append(_dot(pr, va[:, sl]))
            ctx = jnp.concatenate(pieces, axis=1)
            delta = _dot(ctx, wo_ref[...].T) + bo_ref[...]
            out_ref[bi] = pmap_ref[bi] + alpha * delta


def _final_kernel(res_ref, state_ref, og_ref, obn_ref, ow_ref, ob_ref,
                  l2g_ref, l2b_ref, w1_ref, b1_ref, w2_ref, b2_ref, out_ref):
    srow = _ln_rows(state_ref[0], og_ref[...], obn_ref[...])
    orow = _dot(srow, ow_ref[...].T) + ob_ref[...]
    x = res_ref[0] + orow
    u = _ln_rows(x, l2g_ref[...], l2b_ref[...])
    m1 = _gelu(_dot(u.astype(jnp.bfloat16), w1_ref[...].T) + b1_ref[...])
    out_ref[0] = x + _dot(m1.astype(jnp.bfloat16), w2_ref[...].T) + b2_ref[...]


def kernel(x_sacc, x_full, params):
    p = params
    B, N, _ = x_sacc.shape
    nb = N // BLK
    r1 = lambda v: v.reshape(1, -1)

    gpb = 8
    xf_rows = x_full.reshape(B * N, BD)
    xmid, stdv, maxv = pl.pallas_call(
        functools.partial(_periph_stage1_kernel, gpb=gpb),
        grid=(B * nb // gpb,),
        in_specs=[
            pl.BlockSpec((gpb * BLK, BD), lambda i: (i, 0)),
            pl.BlockSpec((256, BD), lambda i: (0, 0)),
            pl.BlockSpec((256, 1), lambda i: (0, 0)),
        ],
        out_specs=[
            pl.BlockSpec((gpb, 256, BLK), lambda i: (i, 0, 0)),
            pl.BlockSpec((gpb, 1, BD), lambda i: (i, 0, 0)),
            pl.BlockSpec((gpb, 1, BD), lambda i: (i, 0, 0)),
        ],
        out_shape=[
            jax.ShapeDtypeStruct((B * nb, 256, BLK), jnp.float32),
            jax.ShapeDtypeStruct((B * nb, 1, BD), jnp.float32),
            jax.ShapeDtypeStruct((B * nb, 1, BD), jnp.float32),
        ],
    )(xf_rows, p['p_conv_proj_w'], p['p_conv_proj_b'].reshape(256, 1))

    conv_in = xmid.reshape(B * nb, 256 * BLK)
    wflat = p['p_conv_w'].reshape(256, 256 * BLK)
    pos = jnp.tile(p['p_pos'][:nb], (B, 1))
    fw, fb = p['f_in_w'], p['f_in_b']
    bqc = fb[:D].reshape(D, 1)
    mask = (lax.broadcasted_iota(jnp.int32, (D, H), 0) // DH ==
            lax.broadcasted_iota(jnp.int32, (D, H), 1)).astype(jnp.float32)
    expander = mask.T

    ctrl_out_shape = [
        jax.ShapeDtypeStruct((B, nb), jnp.float32),
        jax.ShapeDtypeStruct((B, TOPK), jnp.int32),
        jax.ShapeDtypeStruct((B, TOPK), jnp.float32),
        jax.ShapeDtypeStruct((B, D, H), jnp.float32),
        jax.ShapeDtypeStruct((B, 1, H), jnp.float32),
        jax.ShapeDtypeStruct((B, 1, H), jnp.float32),
        jax.ShapeDtypeStruct((B, D), jnp.float32),
    ]

    (pmap_flat, state, scores, ti, tw, comb, sbias, sst, vs) = pl.pallas_call(
        _periph_ctrl_kernel,
        out_shape=[
            jax.ShapeDtypeStruct((B * nb, D), jnp.float32),
            jax.ShapeDtypeStruct((B, D), jnp.float32),
        ] + ctrl_out_shape,
    )(conv_in, wflat, r1(p['p_conv_b']), stdv.reshape(B * nb, BD),
      p['p_std_w'], r1(p['p_std_b']), maxv.reshape(B * nb, BD),
      p['p_max_w'], r1(p['p_max_b']), p['p_proj_w'], r1(p['p_proj_b']),
      r1(p['p_norm_g']), r1(p['p_norm_b']), pos,
      p['c_q_w'], r1(p['c_q_b']), p['c_k_w'], r1(p['c_k_b']),
      r1(p['f_n1_g']), r1(p['f_n1_b']), fw, r1(fb), bqc, mask)
    pmap = pmap_flat.reshape(B, nb, D)

    bm = 512
    ng = B * N // bm
    nbb = ng // B
    x_rows = x_sacc.reshape(B * N, D)
    vtab, sall = pl.pallas_call(
        _kv_score_kernel,
        grid=(ng,),
        in_specs=[pl.BlockSpec((bm, D), lambda i: (i, 0))] +
                 [pl.BlockSpec((1, D), lambda i: (0, 0))] * 4 +
                 [pl.BlockSpec((D, D), lambda i: (0, 0)),
                  pl.BlockSpec((1, D), lambda i: (0, 0)),
                  pl.BlockSpec((1, D, H), lambda i: (i // nbb, 0, 0)),
                  pl.BlockSpec((1, 1, H), lambda i: (i // nbb, 0, 0))],
        out_specs=[pl.BlockSpec((bm, D), lambda i: (i, 0)),
                   pl.BlockSpec((bm, H), lambda i: (i, 0))],
        out_shape=[jax.ShapeDtypeStruct((B * N, D), jnp.float32),
                   jax.ShapeDtypeStruct((B * N, H), jnp.float32)],
    )(x_rows, r1(p['ln1_g']), r1(p['ln1_b']), r1(p['f_n1_g']),
      r1(p['f_n1_b']), fw[2 * D:], r1(fb[2 * D:]), comb, sbias)
    vt3 = vtab.reshape(B, N, D)

    controller = pl.pallas_call(_controller_kernel, out_shape=ctrl_out_shape)
    score_call = pl.pallas_call(
        _score_kernel,
        grid=(ng,),
        in_specs=[pl.BlockSpec((bm, D), lambda i: (i, 0))] +
                 [pl.BlockSpec((1, D), lambda i: (0, 0))] * 4 +
                 [pl.BlockSpec((1, D, H), lambda i: (i // nbb, 0, 0)),
                  pl.BlockSpec((1, 1, H), lambda i: (i // nbb, 0, 0))],
        out_specs=pl.BlockSpec((bm, H), lambda i: (i, 0)),
        out_shape=jax.ShapeDtypeStruct((B * N, H), jnp.float32),
    )

    fps, flogits = [], []
    acc_starts = []
    for t in range(NSACC):
        if t > 0:
            scores, ti, tw, comb, sbias, sst, vs = controller(
                state, pmap, p['c_q_w'], r1(p['c_q_b']), p['c_k_w'],
                r1(p['c_k_b']), r1(p['f_n1_g']), r1(p['f_n1_b']), fw, r1(fb),
                bqc, mask)
            sall = score_call(x_rows, r1(p['ln1_g']), r1(p['ln1_b']),
                              r1(p['f_n1_g']), r1(p['f_n1_b']), comb, sbias)
        fps.append(ti[:, 0] * BLK)
        flogits.append(scores)
        starts = jnp.clip(ti * BLK - WS // 2, 0, N - WS) // 8
        sall3 = sall.reshape(B, N, H)

        astack = (jnp.stack(acc_starts, axis=1) if acc_starts
                  else jnp.zeros((B, 1), jnp.int32))
        nacc = len(acc_starts)

        ctxv = pl.pallas_call(
            functools.partial(_foveal_attn_kernel, nacc=nacc),
            grid=(B,),
            in_specs=[pl.BlockSpec(memory_space=pltpu.SMEM),
                      pl.BlockSpec(memory_space=pltpu.SMEM),
                      pl.BlockSpec((1, N, H), lambda b: (b, 0, 0)),
                      pl.BlockSpec((1, N, D), lambda b: (b, 0, 0)),
                      pl.BlockSpec((1, 1, H), lambda b: (b, 0, 0)),
                      pl.BlockSpec((1, 1, D), lambda b: (b, 0, 0)),
                      pl.BlockSpec((H, D), lambda b: (0, 0))],
            out_specs=pl.BlockSpec((1, TOPK, D), lambda b: (b, 0, 0)),
            out_shape=jax.ShapeDtypeStruct((B, TOPK, D), jnp.float32),
        )(starts, astack, sall3, vt3, sst, vs.reshape(B, 1, D), expander)
        ctxv32 = ctxv.reshape(B * TOPK, D)

        acc_starts.append(starts[:, 0])
        astack2 = jnp.stack(acc_starts, axis=1)

        tt = jnp.array([[t / NSACC]], dtype=jnp.float32)
        a1 = _gelu(tt @ p['g1_w'].T + p['g1_b'])
        alpha = jax.nn.sigmoid(a1 @ p['g2_w'].T + p['g2_b'])

        st32 = jnp.repeat(state, TOPK, axis=0)
        state = pl.pallas_call(
            _dense_kernel,
            grid=(4,),
            in_specs=[
                pl.BlockSpec((B * TOPK, D), lambda j: (0, 0)),
                pl.BlockSpec((B * TOPK, D), lambda j: (0, 0)),
                pl.BlockSpec((B, TOPK), lambda j: (0, 0)),
                pl.BlockSpec((D, D), lambda j: (0, 0)),
                pl.BlockSpec((1, D), lambda j: (0, 0)),
                pl.BlockSpec((1, D), lambda j: (0, 0)),
                pl.BlockSpec((1, D), lambda j: (0, 0)),
                pl.BlockSpec((D, D), lambda j: (j, 0)),
                pl.BlockSpec((1, D), lambda j: (0, j)),
                pl.BlockSpec((D, D), lambda j: (0, j)),
                pl.BlockSpec((1, D), lambda j: (0, 0)),
            ],
            out_specs=pl.BlockSpec((B, D), lambda j: (0, 0)),
            out_shape=jax.ShapeDtypeStruct((B, D), jnp.float32),
            scratch_shapes=[pltpu.VMEM((B * TOPK, D), jnp.float32)],
        )(ctxv32, st32, tw, p['f_out_w'], r1(p['f_out_b']),
          r1(p['f_n2_g']), r1(p['f_n2_b']), p['f_ffn1_w'], r1(p['f_ffn1_b']),
          p['f_ffn2_w'], r1(p['f_ffn2_b']))

        pmap = pl.pallas_call(
            functools.partial(_mattn_kernel, nacc=t + 1),
            grid=(3,),
            in_specs=[pl.BlockSpec(memory_space=pltpu.SMEM),
                      pl.BlockSpec(memory_space=pltpu.SMEM),
                      pl.BlockSpec((B, nb, D), lambda j: (0, 0, 0)),
                      pl.BlockSpec((1, D), lambda j: (0, 0)),
                      pl.BlockSpec((1, D), lambda j: (0, 0)),
                      pl.BlockSpec((D, D), lambda j: (j, 0)),
                      pl.BlockSpec((1, D), lambda j: (0, j)),
                      pl.BlockSpec((D, D), lambda j: (0, 0)),
                      pl.BlockSpec((1, D), lambda j: (0, 0)),
                      pl.BlockSpec((1, D), lambda j: (0, 0)),
                      pl.BlockSpec((1, D), lambda j: (0, 0)),
                      pl.BlockSpec(memory_space=pl.ANY)],
            out_specs=pl.BlockSpec((B, nb, D), lambda j: (0, 0, 0)),
            out_shape=jax.ShapeDtypeStruct((B, nb, D), jnp.float32),
            scratch_shapes=[pltpu.VMEM((B, nb, D), jnp.float32),
                            pltpu.VMEM((B, (t + 1) * WS, D), jnp.float32),
                            pltpu.VMEM((B, (t + 1) * WS, D), jnp.float32),
                            pltpu.VMEM((B, (t + 1) * WS, D), jnp.float32),
                            pltpu.SemaphoreType.DMA],
        )(astack2, alpha, pmap, r1(p['m_norm_g']), r1(p['m_norm_b']),
          p['m_in_w'], r1(p['m_in_b']), p['m_out_w'], r1(p['m_out_b']),
          r1(p['ln1_g']), r1(p['ln1_b']), x_sacc)

    bm2 = 512
    out = pl.pallas_call(
        _final_kernel,
        grid=(B, N // bm2),
        in_specs=[
            pl.BlockSpec((1, bm2, D), lambda b, i: (b, i, 0)),
            pl.BlockSpec((1, 1, D), lambda b, i: (b, 0, 0)),
        ] + [pl.BlockSpec((1, D), lambda b, i: (0, 0))] * 2 + [
            pl.BlockSpec((D, D), lambda b, i: (0, 0)),
            pl.BlockSpec((1, D), lambda b, i: (0, 0)),
            pl.BlockSpec((1, D), lambda b, i: (0, 0)),
            pl.BlockSpec((1, D), lambda b, i: (0, 0)),
            pl.BlockSpec((4 * D, D), lambda b, i: (0, 0)),
            pl.BlockSpec((1, 4 * D), lambda b, i: (0, 0)),
            pl.BlockSpec((D, 4 * D), lambda b, i: (0, 0)),
            pl.BlockSpec((1, D), lambda b, i: (0, 0)),
        ],
        out_specs=pl.BlockSpec((1, bm2, D), lambda b, i: (b, i, 0)),
        out_shape=jax.ShapeDtypeStruct((B, N, D), jnp.float32),
    )(x_sacc, state.reshape(B, 1, D), r1(p['o_norm_g']), r1(p['o_norm_b']),
      p['o_w'], r1(p['o_b']), r1(p['ln2_g']), r1(p['ln2_b']),
      p['mlp1_w'].astype(jnp.bfloat16), r1(p['mlp1_b']),
      p['mlp2_w'].astype(jnp.bfloat16), r1(p['mlp2_b']))

    return out, jnp.stack(fps), jnp.stack(flogits)


# speedup vs baseline: 1.5575x; 1.0035x over previous
import functools
import math

import jax
import jax.numpy as jnp
from jax import lax
from jax.experimental import pallas as pl
from jax.experimental.pallas import tpu as pltpu
from jax.experimental.pallas import tpu_sc as plsc

D = 1024
BD = 2048
H = 16
DH = 64
BLK = 64
WS = 128
NSACC = 2
TOPK = 16


def _dot(a, b):
    return jnp.dot(a, b, preferred_element_type=jnp.float32)


def _dotg(a, b, dims):
    return lax.dot_general(a, b, (dims, ((), ())),
                           preferred_element_type=jnp.float32)


def _ln_rows(x, g, b, eps=1e-5):
    m = jnp.mean(x, axis=-1, keepdims=True)
    v = jnp.mean((x - m) ** 2, axis=-1, keepdims=True)
    return (x - m) * lax.rsqrt(v + eps) * g + b


def _gelu(x):
    return 0.5 * x * (1.0 + lax.erf(x * (2.0 ** -0.5)))


def _sc_topk_call(scores):
    B, nb = scores.shape
    mesh = plsc.VectorSubcoreMesh(core_axis_name="c", subcore_axis_name="s")

    def _bmax(x):
        return plsc.cummax(lax.rev(plsc.cummax(x), (0,)))

    @functools.partial(
        pl.kernel, mesh=mesh,
        out_type=[jax.ShapeDtypeStruct((B, TOPK), jnp.int32),
                  jax.ShapeDtypeStruct((B, TOPK), jnp.float32)],
        scratch_types=[pltpu.VMEM((nb,), jnp.float32),
                       pltpu.VMEM((TOPK,), jnp.int32),
                       pltpu.VMEM((TOPK,), jnp.float32)],
        compiler_params=pltpu.CompilerParams(needs_layout_passes=False),
    )
    def topk_kernel(scores_hbm, ti_hbm, tw_hbm, sc_v, ti_v, tw_v):
        wid = lax.axis_index("s") * 2 + lax.axis_index("c")

        @pl.when(wid == 0)
        def _():
            iota = lax.iota(jnp.int32, 16)
            for bi in range(B):
                pltpu.sync_copy(scores_hbm.at[bi], sc_v)
                v0 = sc_v[pl.ds(0, 16)]
                v1 = sc_v[pl.ds(16, 16)]
                tvv = jnp.zeros((16,), jnp.float32)
                tiv = jnp.zeros((16,), jnp.int32)
                neg = jnp.float32(-jnp.inf)
                for r in range(TOPK):
                    m = _bmax(jnp.maximum(v0, v1))
                    i0 = jnp.where(v0 == m, iota, nb)
                    i1 = jnp.where(v1 == m, iota + 16, nb)
                    idx = -_bmax(-jnp.minimum(i0, i1))
                    tvv = jnp.where(iota == r, m, tvv)
                    tiv = jnp.where(iota == r, idx, tiv)
                    v0 = jnp.where(iota == idx, neg, v0)
                    v1 = jnp.where(iota + 16 == idx, neg, v1)
                e = jnp.exp((tvv - _bmax(tvv)) / 5.0)
                tww = e / plsc.cummax(lax.rev(plsc.cumsum(e), (0,)))
                ti_v[...] = tiv
                tw_v[...] = tww
                pltpu.sync_copy(ti_v, ti_hbm.at[bi])
                pltpu.sync_copy(tw_v, tw_hbm.at[bi])

    return topk_kernel(scores)


def _ctrl_compute(state, pm2, cqw, cqb, ckw, ckb, n1g, n1b, fw, fb, bqc,
                  mask):
    B = state.shape[0]
    nb = pm2.shape[0] // B
    q = _dot(state, cqw.T) + cqb
    kk = _dot(pm2, ckw.T) + ckb
    sfull = _dot(q, kk.T) / math.sqrt(D)
    rows = [sfull[bi:bi + 1, bi * nb:(bi + 1) * nb] for bi in range(B)]
    scores = jnp.concatenate(rows, axis=0)

    g = _ln_rows(state, n1g, n1b)
    wq, wk = fw[:D, :], fw[D:2 * D, :]
    bk, bv = fb[0:1, D:2 * D], fb[0:1, 2 * D:]
    qfT = _dotg(wq, g, ((1,), (1,))) + bqc
    ks = _dot(g, wk.T) + bk
    vstate = _dot(g, fw[2 * D:, :].T) + bv
    combs, sbias, sst = [], [], []
    for bi in range(B):
        qmat = mask * qfT[:, bi:bi + 1]
        combs.append(_dotg(wk, qmat, ((0,), (0,))) / 8.0)
        sbias.append(_dot(bk, qmat) / 8.0)
        sst.append(_dot(ks[bi:bi + 1, :], qmat) / 8.0)
    return scores, combs, sbias, sst, vstate


def _write_ctrl(refs, vals):
    (scores_ref, comb_ref, sb_ref, sst_ref, vs_ref) = refs
    scores, combs, sbias, sst, vstate = vals
    scores_ref[...] = scores
    for bi in range(len(combs)):
        comb_ref[bi] = combs[bi]
        sb_ref[bi] = sbias[bi]
        sst_ref[bi] = sst[bi]
    vs_ref[...] = vstate


def _periph_stage1_kernel(x_ref, wc_ref, bc_ref, xmid_ref, std_ref, max_ref,
                          *, gpb):
    x = x_ref[...]
    xmT = _dotg(wc_ref[...], x, ((1,), (1,))) + bc_ref[...]
    for g in range(gpb):
        xmid_ref[g] = xmT[:, g * BLK:(g + 1) * BLK]
        xs = x[g * BLK:(g + 1) * BLK]
        mean = jnp.mean(xs, axis=0, keepdims=True)
        var = jnp.sum((xs - mean) ** 2, axis=0, keepdims=True) / (BLK - 1)
        std_ref[g] = jnp.sqrt(var)
        max_ref[g] = jnp.max(xs, axis=0, keepdims=True)


def _periph_ctrl_kernel(cin_ref, wf_ref, cb_ref, sv_ref, sw_ref, sb_ref,
                        mv_ref, mw_ref, mb_ref, pp_ref, pb_ref, g_ref, b_ref,
                        pos_ref, cqw_ref, cqb_ref, ckw_ref, ckb_ref, n1g_ref,
                        n1b_ref, fw_ref, fb_ref, bqc_ref, mask_ref,
                        pmap_ref, state_ref, *ctrl_refs):
    co = _dot(cin_ref[...], wf_ref[...].T) + cb_ref[...]
    so = _dot(sv_ref[...], sw_ref[...].T) + sb_ref[...]
    mo = _dot(mv_ref[...], mw_ref[...].T) + mb_ref[...]
    combined = jnp.concatenate([co, so, mo], axis=1)
    pre = _dot(combined, pp_ref[...].T) + pb_ref[...]
    pmap = _ln_rows(pre, g_ref[...], b_ref[...]) + pos_ref[...]
    pmap_ref[...] = pmap
    B = state_ref.shape[0]
    nb = pmap.shape[0] // B
    srows = [jnp.mean(pmap[bi * nb:(bi + 1) * nb], axis=0, keepdims=True)
             for bi in range(B)]
    state = jnp.concatenate(srows, axis=0)
    state_ref[...] = state
    vals = _ctrl_compute(state, pmap, cqw_ref[...], cqb_ref[...], ckw_ref[...],
                         ckb_ref[...], n1g_ref[...], n1b_ref[...], fw_ref[...],
                         fb_ref[...], bqc_ref[...], mask_ref[...])
    _write_ctrl(ctrl_refs, vals)


def _controller_kernel(state_ref, pmap_ref, cqw_ref, cqb_ref, ckw_ref,
                       ckb_ref, n1g_ref, n1b_ref, fw_ref, fb_ref, bqc_ref,
                       mask_ref, *ctrl_refs):
    B = state_ref.shape[0]
    nb = pmap_ref.shape[1]
    pm2 = pmap_ref[...].reshape(B * nb, D)
    vals = _ctrl_compute(state_ref[...], pm2, cqw_ref[...], cqb_ref[...],
                         ckw_ref[...], ckb_ref[...], n1g_ref[...],
                         n1b_ref[...], fw_ref[...], fb_ref[...], bqc_ref[...],
                         mask_ref[...])
    _write_ctrl(ctrl_refs, vals)


def _kv_score_kernel(x_ref, l1g_ref, l1b_ref, n1g_ref, n1b_ref, wv_ref,
                     bv_ref, comb_ref, sb_ref, vt_ref, sall_ref):
    h = _ln_rows(x_ref[...], l1g_ref[...], l1b_ref[...])
    g = _ln_rows(h, n1g_ref[...], n1b_ref[...])
    vt_ref[...] = _dot(g, wv_ref[...].T) + bv_ref[...]
    sall_ref[...] = _dot(g, comb_ref[0]) + sb_ref[0]


def _score_kernel(x_ref, l1g_ref, l1b_ref, n1g_ref, n1b_ref, comb_ref,
                  sb_ref, sall_ref):
    h = _ln_rows(x_ref[...], l1g_ref[...], l1b_ref[...])
    g = _ln_rows(h, n1g_ref[...], n1b_ref[...])
    sall_ref[...] = _dot(g, comb_ref[0]) + sb_ref[0]


def _foveal_attn_kernel(starts_ref, astarts_ref, sall_ref, vt_ref, sst_ref,
                        vs_ref, ex_ref, ctxv_ref, *, nacc):
    b = pl.program_id(0)
    ex = ex_ref[...]
    s_state = sst_ref[0]
    vs = vs_ref[0]

    saccs, vaccs = [], []
    for j in range(nacc):
        a0 = astarts_ref[b, j] * 8
        saccs.append(sall_ref[0, pl.ds(a0, WS), :])
        vaccs.append(vt_ref[0, pl.ds(a0, WS), :])

    for k in range(TOPK):
        st = starts_ref[b, k] * 8
        sw = sall_ref[0, pl.ds(st, WS), :]
        vwin = vt_ref[0, pl.ds(st, WS), :]
        m = jnp.maximum(jnp.max(sw, axis=0, keepdims=True), s_state)
        for sa in saccs:
            m = jnp.maximum(m, jnp.max(sa, axis=0, keepdims=True))
        ew = jnp.exp(sw - m)
        es = jnp.exp(s_state - m)
        denom = jnp.sum(ew, axis=0, keepdims=True) + es
        eas = []
        for sa in saccs:
            ea = jnp.exp(sa - m)
            eas.append(ea)
            denom = denom + jnp.sum(ea, axis=0, keepdims=True)
        inv = 1.0 / denom
        ctxv = jnp.sum(vwin * _dot(ew * inv, ex), axis=0, keepdims=True)
        ctxv = ctxv + vs * _dot(es * inv, ex)
        for ea, vacc in zip(eas, vaccs):
            ctxv = ctxv + jnp.sum(vacc * _dot(ea * inv, ex), axis=0,
                                  keepdims=True)
        ctxv_ref[0, k:k + 1, :] = ctxv


def _dense_kernel(ctxv_ref, st32_ref, tw_ref, ow_ref, ob_ref, n2g_ref,
                  n2b_ref, w1_ref, b1_ref, w2_ref, b2_ref, wst_ref, acc_ref):
    j = pl.program_id(0)
    nj = pl.num_programs(0)
    B, K = tw_ref.shape
    s = st32_ref[...] + _dot(ctxv_ref[...], ow_ref[...].T) + ob_ref[...]
    u = _ln_rows(s, n2g_ref[...], n2b_ref[...])
    m1 = _gelu(_dot(u, w1_ref[...].T) + b1_ref[...])
    pj = _dot(m1, w2_ref[...].T)

    @pl.when(j == 0)
    def _init():
        acc_ref[...] = pj

    @pl.when(j > 0)
    def _acc():
        acc_ref[...] = acc_ref[...] + pj

    @pl.when(j == nj - 1)
    def _fin():
        s2 = s + acc_ref[...] + b2_ref[...]
        for bi in range(B):
            wst_ref[bi:bi + 1, :] = _dot(tw_ref[bi:bi + 1, :],
                                         s2[bi * K:(bi + 1) * K, :])


def _mattn_kernel(astarts_ref, alpha_ref, pmap_ref, ng_ref, nbg_ref, mw_ref,
                  mb_ref, wo_ref, bo_ref, l1g_ref, l1b_ref, x_hbm,
                  out_ref, qs_ref, ka_ref, va_ref, accx_ref, sem, *, nacc):
    j = pl.program_id(0)
    B = pmap_ref.shape[0]

    @pl.when(j == 0)
    def _dma():
        for bi in range(B):
            for jj in range(nacc):
                a0 = astarts_ref[bi, jj] * 8
                cp = pltpu.make_async_copy(
                    x_hbm.at[bi, pl.ds(a0, WS), :],
                    accx_ref.at[bi, pl.ds(jj * WS, WS), :], sem)
                cp.start()
                cp.wait()

    alpha = alpha_ref[0, 0]

    @pl.when(j == 0)
    def _q():
        for bi in range(B):
            pn = _ln_rows(pmap_ref[bi], ng_ref[...], nbg_ref[...])
            qs_ref[bi] = _dot(pn, mw_ref[...].T) + mb_ref[...]

    @pl.when(j == 1)
    def _k():
        for bi in range(B):
            a = _ln_rows(accx_ref[bi], l1g_ref[...], l1b_ref[...])
            ka_ref[bi] = _dot(a, mw_ref[...].T) + mb_ref[...]

    @pl.when(j == 2)
    def _v():
        for bi in range(B):
            a = _ln_rows(accx_ref[bi], l1g_ref[...], l1b_ref[...])
            va_ref[bi] = _dot(a, mw_ref[...].T) + mb_ref[...]
        for bi in range(B):
            q = qs_ref[bi]
            ka = ka_ref[bi]
            va = va_ref[bi]
            pieces = []
            for hh in range(H):
                sl = slice(hh * DH, (hh + 1) * DH)
                sc = _dot(q[:, sl], ka[:, sl].T) / 8.0
                sc = sc - jnp.max(sc, axis=1, keepdims=True)
                pr = jnp.exp(sc)
                pr = pr / jnp.sum(pr, axis=1, keepdims=True)
                pieces.append(_dot(pr, va[:, sl]))
            ctx = jnp.concatenate(pieces, axis=1)
            delta = _dot(ctx, wo_ref[...].T) + bo_ref[...]
            out_ref[bi] = pmap_ref[bi] + alpha * delta


def _final_kernel(res_ref, state_ref, og_ref, obn_ref, ow_ref, ob_ref,
                  l2g_ref, l2b_ref, w1_ref, b1_ref, w2_ref, b2_ref, out_ref):
    srow = _ln_rows(state_ref[0], og_ref[...], obn_ref[...])
    orow = _dot(srow, ow_ref[...].T) + ob_ref[...]
    x = res_ref[0] + orow
    u = _ln_rows(x, l2g_ref[...], l2b_ref[...])
    m1 = _gelu(_dot(u.astype(jnp.bfloat16), w1_ref[...].T) + b1_ref[...])
    out_ref[0] = x + _dot(m1.astype(jnp.bfloat16), w2_ref[...].T) + b2_ref[...]


def kernel(x_sacc, x_full, params):
    p = params
    B, N, _ = x_sacc.shape
    nb = N // BLK
    r1 = lambda v: v.reshape(1, -1)

    gpb = 8
    xf_rows = x_full.reshape(B * N, BD)
    xmid, stdv, maxv = pl.pallas_call(
        functools.partial(_periph_stage1_kernel, gpb=gpb),
        grid=(B * nb // gpb,),
        in_specs=[
            pl.BlockSpec((gpb * BLK, BD), lambda i: (i, 0)),
            pl.BlockSpec((256, BD), lambda i: (0, 0)),
            pl.BlockSpec((256, 1), lambda i: (0, 0)),
        ],
        out_specs=[
            pl.BlockSpec((gpb, 256, BLK), lambda i: (i, 0, 0)),
            pl.BlockSpec((gpb, 1, BD), lambda i: (i, 0, 0)),
            pl.BlockSpec((gpb, 1, BD), lambda i: (i, 0, 0)),
        ],
        out_shape=[
            jax.ShapeDtypeStruct((B * nb, 256, BLK), jnp.float32),
            jax.ShapeDtypeStruct((B * nb, 1, BD), jnp.float32),
            jax.ShapeDtypeStruct((B * nb, 1, BD), jnp.float32),
        ],
    )(xf_rows, p['p_conv_proj_w'], p['p_conv_proj_b'].reshape(256, 1))

    conv_in = xmid.reshape(B * nb, 256 * BLK)
    wflat = p['p_conv_w'].reshape(256, 256 * BLK)
    pos = jnp.tile(p['p_pos'][:nb], (B, 1))
    fw, fb = p['f_in_w'], p['f_in_b']
    bqc = fb[:D].reshape(D, 1)
    mask = (lax.broadcasted_iota(jnp.int32, (D, H), 0) // DH ==
            lax.broadcasted_iota(jnp.int32, (D, H), 1)).astype(jnp.float32)
    expander = mask.T

    ctrl_out_shape = [
        jax.ShapeDtypeStruct((B, nb), jnp.float32),
        jax.ShapeDtypeStruct((B, D, H), jnp.float32),
        jax.ShapeDtypeStruct((B, 1, H), jnp.float32),
        jax.ShapeDtypeStruct((B, 1, H), jnp.float32),
        jax.ShapeDtypeStruct((B, D), jnp.float32),
    ]

    (pmap_flat, state, scores, comb, sbias, sst, vs) = pl.pallas_call(
        _periph_ctrl_kernel,
        out_shape=[
            jax.ShapeDtypeStruct((B * nb, D), jnp.float32),
            jax.ShapeDtypeStruct((B, D), jnp.float32),
        ] + ctrl_out_shape,
    )(conv_in, wflat, r1(p['p_conv_b']), stdv.reshape(B * nb, BD),
      p['p_std_w'], r1(p['p_std_b']), maxv.reshape(B * nb, BD),
      p['p_max_w'], r1(p['p_max_b']), p['p_proj_w'], r1(p['p_proj_b']),
      r1(p['p_norm_g']), r1(p['p_norm_b']), pos,
      p['c_q_w'], r1(p['c_q_b']), p['c_k_w'], r1(p['c_k_b']),
      r1(p['f_n1_g']), r1(p['f_n1_b']), fw, r1(fb), bqc, mask)
    pmap = pmap_flat.reshape(B, nb, D)

    bm = 512
    ng = B * N // bm
    nbb = ng // B
    x_rows = x_sacc.reshape(B * N, D)
    vtab, sall = pl.pallas_call(
        _kv_score_kernel,
        grid=(ng,),
        in_specs=[pl.BlockSpec((bm, D), lambda i: (i, 0))] +
                 [pl.BlockSpec((1, D), lambda i: (0, 0))] * 4 +
                 [pl.BlockSpec((D, D), lambda i: (0, 0)),
                  pl.BlockSpec((1, D), lambda i: (0, 0)),
                  pl.BlockSpec((1, D, H), lambda i: (i // nbb, 0, 0)),
                  pl.BlockSpec((1, 1, H), lambda i: (i // nbb, 0, 0))],
        out_specs=[pl.BlockSpec((bm, D), lambda i: (i, 0)),
                   pl.BlockSpec((bm, H), lambda i: (i, 0))],
        out_shape=[jax.ShapeDtypeStruct((B * N, D), jnp.float32),
                   jax.ShapeDtypeStruct((B * N, H), jnp.float32)],
    )(x_rows, r1(p['ln1_g']), r1(p['ln1_b']), r1(p['f_n1_g']),
      r1(p['f_n1_b']), fw[2 * D:], r1(fb[2 * D:]), comb, sbias)
    vt3 = vtab.reshape(B, N, D)

    controller = pl.pallas_call(_controller_kernel, out_shape=ctrl_out_shape)
    score_call = pl.pallas_call(
        _score_kernel,
        grid=(ng,),
        in_specs=[pl.BlockSpec((bm, D), lambda i: (i, 0))] +
                 [pl.BlockSpec((1, D), lambda i: (0, 0))] * 4 +
                 [pl.BlockSpec((1, D, H), lambda i: (i // nbb, 0, 0)),
                  pl.BlockSpec((1, 1, H), lambda i: (i // nbb, 0, 0))],
        out_specs=pl.BlockSpec((bm, H), lambda i: (i, 0)),
        out_shape=jax.ShapeDtypeStruct((B * N, H), jnp.float32),
    )

    fps, flogits = [], []
    acc_starts = []
    for t in range(NSACC):
        if t > 0:
            scores, comb, sbias, sst, vs = controller(
                state, pmap, p['c_q_w'], r1(p['c_q_b']), p['c_k_w'],
                r1(p['c_k_b']), r1(p['f_n1_g']), r1(p['f_n1_b']), fw, r1(fb),
                bqc, mask)
            sall = score_call(x_rows, r1(p['ln1_g']), r1(p['ln1_b']),
                              r1(p['f_n1_g']), r1(p['f_n1_b']), comb, sbias)
        ti, tw = _sc_topk_call(scores)
        fps.append(ti[:, 0] * BLK)
        flogits.append(scores)
        starts = jnp.clip(ti * BLK - WS // 2, 0, N - WS) // 8
        sall3 = sall.reshape(B, N, H)

        astack = (jnp.stack(acc_starts, axis=1) if acc_starts
                  else jnp.zeros((B, 1), jnp.int32))
        nacc = len(acc_starts)

        ctxv = pl.pallas_call(
            functools.partial(_foveal_attn_kernel, nacc=nacc),
            grid=(B,),
            in_specs=[pl.BlockSpec(memory_space=pltpu.SMEM),
                      pl.BlockSpec(memory_space=pltpu.SMEM),
                      pl.BlockSpec((1, N, H), lambda b: (b, 0, 0)),
                      pl.BlockSpec((1, N, D), lambda b: (b, 0, 0)),
                      pl.BlockSpec((1, 1, H), lambda b: (b, 0, 0)),
                      pl.BlockSpec((1, 1, D), lambda b: (b, 0, 0)),
                      pl.BlockSpec((H, D), lambda b: (0, 0))],
            out_specs=pl.BlockSpec((1, TOPK, D), lambda b: (b, 0, 0)),
            out_shape=jax.ShapeDtypeStruct((B, TOPK, D), jnp.float32),
        )(starts, astack, sall3, vt3, sst, vs.reshape(B, 1, D), expander)
        ctxv32 = ctxv.reshape(B * TOPK, D)

        acc_starts.append(starts[:, 0])
        astack2 = jnp.stack(acc_starts, axis=1)

        tt = jnp.array([[t / NSACC]], dtype=jnp.float32)
        a1 = _gelu(tt @ p['g1_w'].T + p['g1_b'])
        alpha = jax.nn.sigmoid(a1 @ p['g2_w'].T + p['g2_b'])

        st32 = jnp.repeat(state, TOPK, axis=0)
        state = pl.pallas_call(
            _dense_kernel,
            grid=(4,),
            in_specs=[
                pl.BlockSpec((B * TOPK, D), lambda j: (0, 0)),
                pl.BlockSpec((B * TOPK, D), lambda j: (0, 0)),
                pl.BlockSpec((B, TOPK), lambda j: (0, 0)),
                pl.BlockSpec((D, D), lambda j: (0, 0)),
                pl.BlockSpec((1, D), lambda j: (0, 0)),
                pl.BlockSpec((1, D), lambda j: (0, 0)),
                pl.BlockSpec((1, D), lambda j: (0, 0)),
                pl.BlockSpec((D, D), lambda j: (j, 0)),
                pl.BlockSpec((1, D), lambda j: (0, j)),
                pl.BlockSpec((D, D), lambda j: (0, j)),
                pl.BlockSpec((1, D), lambda j: (0, 0)),
            ],
            out_specs=pl.BlockSpec((B, D), lambda j: (0, 0)),
            out_shape=jax.ShapeDtypeStruct((B, D), jnp.float32),
            scratch_shapes=[pltpu.VMEM((B * TOPK, D), jnp.float32)],
        )(ctxv32, st32, tw, p['f_out_w'], r1(p['f_out_b']),
          r1(p['f_n2_g']), r1(p['f_n2_b']), p['f_ffn1_w'], r1(p['f_ffn1_b']),
          p['f_ffn2_w'], r1(p['f_ffn2_b']))

        pmap = pl.pallas_call(
            functools.partial(_mattn_kernel, nacc=t + 1),
            grid=(3,),
            in_specs=[pl.BlockSpec(memory_space=pltpu.SMEM),
                      pl.BlockSpec(memory_space=pltpu.SMEM),
                      pl.BlockSpec((B, nb, D), lambda j: (0, 0, 0)),
                      pl.BlockSpec((1, D), lambda j: (0, 0)),
                      pl.BlockSpec((1, D), lambda j: (0, 0)),
                      pl.BlockSpec((D, D), lambda j: (j, 0)),
                      pl.BlockSpec((1, D), lambda j: (0, j)),
                      pl.BlockSpec((D, D), lambda j: (0, 0)),
                      pl.BlockSpec((1, D), lambda j: (0, 0)),
                      pl.BlockSpec((1, D), lambda j: (0, 0)),
                      pl.BlockSpec((1, D), lambda j: (0, 0)),
                      pl.BlockSpec(memory_space=pl.ANY)],
            out_specs=pl.BlockSpec((B, nb, D), lambda j: (0, 0, 0)),
            out_shape=jax.ShapeDtypeStruct((B, nb, D), jnp.float32),
            scratch_shapes=[pltpu.VMEM((B, nb, D), jnp.float32),
                            pltpu.VMEM((B, (t + 1) * WS, D), jnp.float32),
                            pltpu.VMEM((B, (t + 1) * WS, D), jnp.float32),
                            pltpu.VMEM((B, (t + 1) * WS, D), jnp.float32),
                            pltpu.SemaphoreType.DMA],
        )(astack2, alpha, pmap, r1(p['m_norm_g']), r1(p['m_norm_b']),
          p['m_in_w'], r1(p['m_in_b']), p['m_out_w'], r1(p['m_out_b']),
          r1(p['ln1_g']), r1(p['ln1_b']), x_sacc)

    bm2 = 512
    out = pl.pallas_call(
        _final_kernel,
        grid=(B, N // bm2),
        in_specs=[
            pl.BlockSpec((1, bm2, D), lambda b, i: (b, i, 0)),
            pl.BlockSpec((1, 1, D), lambda b, i: (b, 0, 0)),
        ] + [pl.BlockSpec((1, D), lambda b, i: (0, 0))] * 2 + [
            pl.BlockSpec((D, D), lambda b, i: (0, 0)),
            pl.BlockSpec((1, D), lambda b, i: (0, 0)),
            pl.BlockSpec((1, D), lambda b, i: (0, 0)),
            pl.BlockSpec((1, D), lambda b, i: (0, 0)),
            pl.BlockSpec((4 * D, D), lambda b, i: (0, 0)),
            pl.BlockSpec((1, 4 * D), lambda b, i: (0, 0)),
            pl.BlockSpec((D, 4 * D), lambda b, i: (0, 0)),
            pl.BlockSpec((1, D), lambda b, i: (0, 0)),
        ],
        out_specs=pl.BlockSpec((1, bm2, D), lambda b, i: (b, i, 0)),
        out_shape=jax.ShapeDtypeStruct((B, N, D), jnp.float32),
    )(x_sacc, state.reshape(B, 1, D), r1(p['o_norm_g']), r1(p['o_norm_b']),
      p['o_w'], r1(p['o_b']), r1(p['ln2_g']), r1(p['ln2_b']),
      p['mlp1_w'].astype(jnp.bfloat16), r1(p['mlp1_b']),
      p['mlp2_w'].astype(jnp.bfloat16), r1(p['mlp2_b']))

    return out, jnp.stack(fps), jnp.stack(flogits)


# speedup vs baseline: 1.5715x; 1.0090x over previous
import functools
import math

import jax
import jax.numpy as jnp
from jax import lax
from jax.experimental import pallas as pl
from jax.experimental.pallas import tpu as pltpu
from jax.experimental.pallas import tpu_sc as plsc

D = 1024
BD = 2048
H = 16
DH = 64
BLK = 64
WS = 128
NSACC = 2
TOPK = 16


def _dot(a, b):
    return jnp.dot(a, b, preferred_element_type=jnp.float32)


def _dotg(a, b, dims):
    return lax.dot_general(a, b, (dims, ((), ())),
                           preferred_element_type=jnp.float32)


def _ln_rows(x, g, b, eps=1e-5):
    m = jnp.mean(x, axis=-1, keepdims=True)
    v = jnp.mean((x - m) ** 2, axis=-1, keepdims=True)
    return (x - m) * lax.rsqrt(v + eps) * g + b


def _gelu(x):
    return 0.5 * x * (1.0 + lax.erf(x * (2.0 ** -0.5)))


def _sc_topk_call(scores):
    B, nb = scores.shape
    mesh = plsc.VectorSubcoreMesh(core_axis_name="c", subcore_axis_name="s")

    def _bmax(x):
        return plsc.cummax(lax.rev(plsc.cummax(x), (0,)))

    @functools.partial(
        pl.kernel, mesh=mesh,
        out_type=[jax.ShapeDtypeStruct((B, TOPK), jnp.int32),
                  jax.ShapeDtypeStruct((B, TOPK), jnp.float32)],
        scratch_types=[pltpu.VMEM((nb,), jnp.float32),
                       pltpu.VMEM((TOPK,), jnp.int32),
                       pltpu.VMEM((TOPK,), jnp.float32)],
        compiler_params=pltpu.CompilerParams(needs_layout_passes=False),
    )
    def topk_kernel(scores_hbm, ti_hbm, tw_hbm, sc_v, ti_v, tw_v):
        wid = lax.axis_index("s") * 2 + lax.axis_index("c")

        @pl.when(wid == 0)
        def _():
            iota = lax.iota(jnp.int32, 16)
            for bi in range(B):
                pltpu.sync_copy(scores_hbm.at[bi], sc_v)
                v0 = sc_v[pl.ds(0, 16)]
                v1 = sc_v[pl.ds(16, 16)]
                tvv = jnp.zeros((16,), jnp.float32)
                tiv = jnp.zeros((16,), jnp.int32)
                neg = jnp.float32(-jnp.inf)
                for r in range(TOPK):
                    m = _bmax(jnp.maximum(v0, v1))
                    i0 = jnp.where(v0 == m, iota, nb)
                    i1 = jnp.where(v1 == m, iota + 16, nb)
                    idx = -_bmax(-jnp.minimum(i0, i1))
                    tvv = jnp.where(iota == r, m, tvv)
                    tiv = jnp.where(iota == r, idx, tiv)
                    v0 = jnp.where(iota == idx, neg, v0)
                    v1 = jnp.where(iota + 16 == idx, neg, v1)
                e = jnp.exp((tvv - _bmax(tvv)) / 5.0)
                tww = e / plsc.cummax(lax.rev(plsc.cumsum(e), (0,)))
                ti_v[...] = tiv
                tw_v[...] = tww
                pltpu.sync_copy(ti_v, ti_hbm.at[bi])
                pltpu.sync_copy(tw_v, tw_hbm.at[bi])

    return topk_kernel(scores)


def _ctrl_compute(state, pm2, cqw, cqb, ckw, ckb, n1g, n1b, fw, fb, bqc,
                  mask):
    B = state.shape[0]
    nb = pm2.shape[0] // B
    q = _dot(state, cqw.T) + cqb
    kk = _dot(pm2, ckw.T) + ckb
    sfull = _dot(q, kk.T) / math.sqrt(D)
    rows = [sfull[bi:bi + 1, bi * nb:(bi + 1) * nb] for bi in range(B)]
    scores = jnp.concatenate(rows, axis=0)

    g = _ln_rows(state, n1g, n1b)
    wq, wk = fw[:D, :], fw[D:2 * D, :]
    bk, bv = fb[0:1, D:2 * D], fb[0:1, 2 * D:]
    qfT = _dotg(wq, g, ((1,), (1,))) + bqc
    ks = _dot(g, wk.T) + bk
    vstate = _dot(g, fw[2 * D:, :].T) + bv
    combs, sbias, sst = [], [], []
    for bi in range(B):
        qmat = mask * qfT[:, bi:bi + 1]
        combs.append(_dotg(wk, qmat, ((0,), (0,))) / 8.0)
        sbias.append(_dot(bk, qmat) / 8.0)
        sst.append(_dot(ks[bi:bi + 1, :], qmat) / 8.0)
    return scores, combs, sbias, sst, vstate


def _write_ctrl(refs, vals):
    (scores_ref, comb_ref, sb_ref, sst_ref, vs_ref) = refs
    scores, combs, sbias, sst, vstate = vals
    scores_ref[...] = scores
    for bi in range(len(combs)):
        comb_ref[bi] = combs[bi]
        sb_ref[bi] = sbias[bi]
        sst_ref[bi] = sst[bi]
    vs_ref[...] = vstate


def _periph_stage1_kernel(x_ref, wc_ref, bc_ref, xmid_ref, std_ref, max_ref,
                          *, gpb):
    x = x_ref[...]
    xmT = _dotg(wc_ref[...], x, ((1,), (1,))) + bc_ref[...]
    for g in range(gpb):
        xmid_ref[g] = xmT[:, g * BLK:(g + 1) * BLK]
        xs = x[g * BLK:(g + 1) * BLK]
        mean = jnp.mean(xs, axis=0, keepdims=True)
        var = jnp.sum((xs - mean) ** 2, axis=0, keepdims=True) / (BLK - 1)
        std_ref[g] = jnp.sqrt(var)
        max_ref[g] = jnp.max(xs, axis=0, keepdims=True)


def _periph_ctrl_kernel(cin_ref, wf_ref, cb_ref, sv_ref, sw_ref, sb_ref,
                        mv_ref, mw_ref, mb_ref, pp_ref, pb_ref, g_ref, b_ref,
                        pos_ref, cqw_ref, cqb_ref, ckw_ref, ckb_ref, n1g_ref,
                        n1b_ref, fw_ref, fb_ref, bqc_ref, mask_ref,
                        pmap_ref, state_ref, *refs):
    ctrl_refs, (co_ref,) = refs[:-1], refs[-1:]
    j = pl.program_id(0)
    nj = pl.num_programs(0)
    pj = _dot(cin_ref[...], wf_ref[...].T)

    @pl.when(j == 0)
    def _init():
        co_ref[...] = pj

    @pl.when(j > 0)
    def _acc():
        co_ref[...] = co_ref[...] + pj

    @pl.when(j == nj - 1)
    def _rest():
        co = co_ref[...] + cb_ref[...]
        so = _dot(sv_ref[...], sw_ref[...].T) + sb_ref[...]
        mo = _dot(mv_ref[...], mw_ref[...].T) + mb_ref[...]
        combined = jnp.concatenate([co, so, mo], axis=1)
        pre = _dot(combined, pp_ref[...].T) + pb_ref[...]
        pmap = _ln_rows(pre, g_ref[...], b_ref[...]) + pos_ref[...]
        pmap_ref[...] = pmap
        B = state_ref.shape[0]
        nb = pmap.shape[0] // B
        srows = [jnp.mean(pmap[bi * nb:(bi + 1) * nb], axis=0, keepdims=True)
                 for bi in range(B)]
        state = jnp.concatenate(srows, axis=0)
        state_ref[...] = state
        vals = _ctrl_compute(state, pmap, cqw_ref[...], cqb_ref[...],
                             ckw_ref[...], ckb_ref[...], n1g_ref[...],
                             n1b_ref[...], fw_ref[...], fb_ref[...],
                             bqc_ref[...], mask_ref[...])
        _write_ctrl(ctrl_refs, vals)


def _controller_kernel(state_ref, pmap_ref, cqw_ref, cqb_ref, ckw_ref,
                       ckb_ref, n1g_ref, n1b_ref, fw_ref, fb_ref, bqc_ref,
                       mask_ref, *ctrl_refs):
    B = state_ref.shape[0]
    nb = pmap_ref.shape[1]
    pm2 = pmap_ref[...].reshape(B * nb, D)
    vals = _ctrl_compute(state_ref[...], pm2, cqw_ref[...], cqb_ref[...],
                         ckw_ref[...], ckb_ref[...], n1g_ref[...],
                         n1b_ref[...], fw_ref[...], fb_ref[...], bqc_ref[...],
                         mask_ref[...])
    _write_ctrl(ctrl_refs, vals)


def _kv_score_kernel(x_ref, l1g_ref, l1b_ref, n1g_ref, n1b_ref, wv_ref,
                     bv_ref, comb_ref, sb_ref, vt_ref, sall_ref):
    h = _ln_rows(x_ref[...], l1g_ref[...], l1b_ref[...])
    g = _ln_rows(h, n1g_ref[...], n1b_ref[...])
    vt_ref[...] = _dot(g, wv_ref[...].T) + bv_ref[...]
    sall_ref[...] = _dot(g, comb_ref[0]) + sb_ref[0]


def _score_kernel(x_ref, l1g_ref, l1b_ref, n1g_ref, n1b_ref, comb_ref,
                  sb_ref, sall_ref):
    h = _ln_rows(x_ref[...], l1g_ref[...], l1b_ref[...])
    g = _ln_rows(h, n1g_ref[...], n1b_ref[...])
    sall_ref[...] = _dot(g, comb_ref[0]) + sb_ref[0]


def _foveal_attn_kernel(starts_ref, astarts_ref, sall_ref, vt_ref, sst_ref,
                        vs_ref, ex_ref, ctxv_ref, *, nacc):
    b = pl.program_id(0)
    ex = ex_ref[...]
    s_state = sst_ref[0]
    vs = vs_ref[0]

    saccs, vaccs = [], []
    for j in range(nacc):
        a0 = astarts_ref[b, j] * 8
        saccs.append(sall_ref[0, pl.ds(a0, WS), :])
        vaccs.append(vt_ref[0, pl.ds(a0, WS), :])

    for k in range(TOPK):
        st = starts_ref[b, k] * 8
        sw = sall_ref[0, pl.ds(st, WS), :]
        vwin = vt_ref[0, pl.ds(st, WS), :]
        m = jnp.maximum(jnp.max(sw, axis=0, keepdims=True), s_state)
        for sa in saccs:
            m = jnp.maximum(m, jnp.max(sa, axis=0, keepdims=True))
        ew = jnp.exp(sw - m)
        es = jnp.exp(s_state - m)
        denom = jnp.sum(ew, axis=0, keepdims=True) + es
        eas = []
        for sa in saccs:
            ea = jnp.exp(sa - m)
            eas.append(ea)
            denom = denom + jnp.sum(ea, axis=0, keepdims=True)
        inv = 1.0 / denom
        ctxv = jnp.sum(vwin * _dot(ew * inv, ex), axis=0, keepdims=True)
        ctxv = ctxv + vs * _dot(es * inv, ex)
        for ea, vacc in zip(eas, vaccs):
            ctxv = ctxv + jnp.sum(vacc * _dot(ea * inv, ex), axis=0,
                                  keepdims=True)
        ctxv_ref[0, k:k + 1, :] = ctxv


def _dense_kernel(ctxv_ref, st32_ref, tw_ref, ow_ref, ob_ref, n2g_ref,
                  n2b_ref, w1_ref, b1_ref, w2_ref, b2_ref, wst_ref, acc_ref):
    j = pl.program_id(0)
    nj = pl.num_programs(0)
    B, K = tw_ref.shape
    s = st32_ref[...] + _dot(ctxv_ref[...], ow_ref[...].T) + ob_ref[...]
    u = _ln_rows(s, n2g_ref[...], n2b_ref[...])
    m1 = _gelu(_dot(u, w1_ref[...].T) + b1_ref[...])
    pj = _dot(m1, w2_ref[...].T)

    @pl.when(j == 0)
    def _init():
        acc_ref[...] = pj

    @pl.when(j > 0)
    def _acc():
        acc_ref[...] = acc_ref[...] + pj

    @pl.when(j == nj - 1)
    def _fin():
        s2 = s + acc_ref[...] + b2_ref[...]
        for bi in range(B):
            wst_ref[bi:bi + 1, :] = _dot(tw_ref[bi:bi + 1, :],
                                         s2[bi * K:(bi + 1) * K, :])


def _mattn_kernel(astarts_ref, alpha_ref, pmap_ref, ng_ref, nbg_ref, mw_ref,
                  mb_ref, wo_ref, bo_ref, l1g_ref, l1b_ref, x_hbm,
                  out_ref, qs_ref, ka_ref, va_ref, accx_ref, sem, *, nacc):
    j = pl.program_id(0)
    B = pmap_ref.shape[0]

    @pl.when(j == 0)
    def _dma():
        for bi in range(B):
            for jj in range(nacc):
                a0 = astarts_ref[bi, jj] * 8
                cp = pltpu.make_async_copy(
                    x_hbm.at[bi, pl.ds(a0, WS), :],
                    accx_ref.at[bi, pl.ds(jj * WS, WS), :], sem)
                cp.start()
                cp.wait()

    alpha = alpha_ref[0, 0]

    @pl.when(j == 0)
    def _q():
        for bi in range(B):
            pn = _ln_rows(pmap_ref[bi], ng_ref[...], nbg_ref[...])
            qs_ref[bi] = _dot(pn, mw_ref[...].T) + mb_ref[...]

    @pl.when(j == 1)
    def _k():
        for bi in range(B):
            a = _ln_rows(accx_ref[bi], l1g_ref[...], l1b_ref[...])
            ka_ref[bi] = _dot(a, mw_ref[...].T) + mb_ref[...]

    @pl.when(j == 2)
    def _v():
        for bi in range(B):
            a = _ln_rows(accx_ref[bi], l1g_ref[...], l1b_ref[...])
            va_ref[bi] = _dot(a, mw_ref[...].T) + mb_ref[...]
        for bi in range(B):
            q = qs_ref[bi]
            ka = ka_ref[bi]
            va = va_ref[bi]
            pieces = []
            for hh in range(H):
                sl = slice(hh * DH, (hh + 1) * DH)
                sc = _dot(q[:, sl], ka[:, sl].T) / 8.0
                sc = sc - jnp.max(sc, axis=1, keepdims=True)
                pr = jnp.exp(sc)
                pr = pr / jnp.sum(pr, axis=1, keepdims=True)
                pieces.append(_dot(pr, va[:, sl]))
            ctx = jnp.concatenate(pieces, axis=1)
            delta = _dot(ctx, wo_ref[...].T) + bo_ref[...]
            out_ref[bi] = pmap_ref[bi] + alpha * delta


def _final_kernel(res_ref, state_ref, og_ref, obn_ref, ow_ref, ob_ref,
                  l2g_ref, l2b_ref, w1_ref, b1_ref, w2_ref, b2_ref, out_ref):
    srow = _ln_rows(state_ref[0], og_ref[...], obn_ref[...])
    orow = _dot(srow, ow_ref[...].T) + ob_ref[...]
    x = res_ref[0] + orow
    u = _ln_rows(x, l2g_ref[...], l2b_ref[...])
    m1 = _gelu(_dot(u.astype(jnp.bfloat16), w1_ref[...].T) + b1_ref[...])
    out_ref[0] = x + _dot(m1.astype(jnp.bfloat16), w2_ref[...].T) + b2_ref[...]


def kernel(x_sacc, x_full, params):
    p = params
    B, N, _ = x_sacc.shape
    nb = N // BLK
    r1 = lambda v: v.reshape(1, -1)

    gpb = 8
    xf_rows = x_full.reshape(B * N, BD)
    xmid, stdv, maxv = pl.pallas_call(
        functools.partial(_periph_stage1_kernel, gpb=gpb),
        grid=(B * nb // gpb,),
        in_specs=[
            pl.BlockSpec((gpb * BLK, BD), lambda i: (i, 0)),
            pl.BlockSpec((256, BD), lambda i: (0, 0)),
            pl.BlockSpec((256, 1), lambda i: (0, 0)),
        ],
        out_specs=[
            pl.BlockSpec((gpb, 256, BLK), lambda i: (i, 0, 0)),
            pl.BlockSpec((gpb, 1, BD), lambda i: (i, 0, 0)),
            pl.BlockSpec((gpb, 1, BD), lambda i: (i, 0, 0)),
        ],
        out_shape=[
            jax.ShapeDtypeStruct((B * nb, 256, BLK), jnp.float32),
            jax.ShapeDtypeStruct((B * nb, 1, BD), jnp.float32),
            jax.ShapeDtypeStruct((B * nb, 1, BD), jnp.float32),
        ],
    )(xf_rows, p['p_conv_proj_w'], p['p_conv_proj_b'].reshape(256, 1))

    conv_in = xmid.reshape(B * nb, 256 * BLK)
    wflat = p['p_conv_w'].reshape(256, 256 * BLK)
    pos = jnp.tile(p['p_pos'][:nb], (B, 1))
    fw, fb = p['f_in_w'], p['f_in_b']
    bqc = fb[:D].reshape(D, 1)
    mask = (lax.broadcasted_iota(jnp.int32, (D, H), 0) // DH ==
            lax.broadcasted_iota(jnp.int32, (D, H), 1)).astype(jnp.float32)
    expander = mask.T

    ctrl_out_shape = [
        jax.ShapeDtypeStruct((B, nb), jnp.float32),
        jax.ShapeDtypeStruct((B, D, H), jnp.float32),
        jax.ShapeDtypeStruct((B, 1, H), jnp.float32),
        jax.ShapeDtypeStruct((B, 1, H), jnp.float32),
        jax.ShapeDtypeStruct((B, D), jnp.float32),
    ]

    kc = 4096
    (pmap_flat, state, scores, comb, sbias, sst, vs) = pl.pallas_call(
        _periph_ctrl_kernel,
        grid=(256 * BLK // kc,),
        in_specs=[pl.BlockSpec((B * nb, kc), lambda j: (0, j)),
                  pl.BlockSpec((256, kc), lambda j: (0, j))] +
                 [pl.BlockSpec((1, 256), lambda j: (0, 0)),
                  pl.BlockSpec((B * nb, BD), lambda j: (0, 0)),
                  pl.BlockSpec((256, BD), lambda j: (0, 0)),
                  pl.BlockSpec((1, 256), lambda j: (0, 0)),
                  pl.BlockSpec((B * nb, BD), lambda j: (0, 0)),
                  pl.BlockSpec((256, BD), lambda j: (0, 0)),
                  pl.BlockSpec((1, 256), lambda j: (0, 0)),
                  pl.BlockSpec((D, 768), lambda j: (0, 0)),
                  pl.BlockSpec((1, D), lambda j: (0, 0)),
                  pl.BlockSpec((1, D), lambda j: (0, 0)),
                  pl.BlockSpec((1, D), lambda j: (0, 0)),
                  pl.BlockSpec((B * nb, D), lambda j: (0, 0)),
                  pl.BlockSpec((D, D), lambda j: (0, 0)),
                  pl.BlockSpec((1, D), lambda j: (0, 0)),
                  pl.BlockSpec((D, D), lambda j: (0, 0)),
                  pl.BlockSpec((1, D), lambda j: (0, 0)),
                  pl.BlockSpec((1, D), lambda j: (0, 0)),
                  pl.BlockSpec((1, D), lambda j: (0, 0)),
                  pl.BlockSpec((3 * D, D), lambda j: (0, 0)),
                  pl.BlockSpec((1, 3 * D), lambda j: (0, 0)),
                  pl.BlockSpec((D, 1), lambda j: (0, 0)),
                  pl.BlockSpec((D, H), lambda j: (0, 0))],
        out_specs=[pl.BlockSpec((B * nb, D), lambda j: (0, 0)),
                   pl.BlockSpec((B, D), lambda j: (0, 0)),
                   pl.BlockSpec((B, nb), lambda j: (0, 0)),
                   pl.BlockSpec((B, D, H), lambda j: (0, 0, 0)),
                   pl.BlockSpec((B, 1, H), lambda j: (0, 0, 0)),
                   pl.BlockSpec((B, 1, H), lambda j: (0, 0, 0)),
                   pl.BlockSpec((B, D), lambda j: (0, 0))],
        out_shape=[
            jax.ShapeDtypeStruct((B * nb, D), jnp.float32),
            jax.ShapeDtypeStruct((B, D), jnp.float32),
        ] + ctrl_out_shape,
        scratch_shapes=[pltpu.VMEM((B * nb, 256), jnp.float32)],
    )(conv_in, wflat, r1(p['p_conv_b']), stdv.reshape(B * nb, BD),
      p['p_std_w'], r1(p['p_std_b']), maxv.reshape(B * nb, BD),
      p['p_max_w'], r1(p['p_max_b']), p['p_proj_w'], r1(p['p_proj_b']),
      r1(p['p_norm_g']), r1(p['p_norm_b']), pos,
      p['c_q_w'], r1(p['c_q_b']), p['c_k_w'], r1(p['c_k_b']),
      r1(p['f_n1_g']), r1(p['f_n1_b']), fw, r1(fb), bqc, mask)
    pmap = pmap_flat.reshape(B, nb, D)

    bm = 512
    ng = B * N // bm
    nbb = ng // B
    x_rows = x_sacc.reshape(B * N, D)
    vtab, sall = pl.pallas_call(
        _kv_score_kernel,
        grid=(ng,),
        in_specs=[pl.BlockSpec((bm, D), lambda i: (i, 0))] +
                 [pl.BlockSpec((1, D), lambda i: (0, 0))] * 4 +
                 [pl.BlockSpec((D, D), lambda i: (0, 0)),
                  pl.BlockSpec((1, D), lambda i: (0, 0)),
                  pl.BlockSpec((1, D, H), lambda i: (i // nbb, 0, 0)),
                  pl.BlockSpec((1, 1, H), lambda i: (i // nbb, 0, 0))],
        out_specs=[pl.BlockSpec((bm, D), lambda i: (i, 0)),
                   pl.BlockSpec((bm, H), lambda i: (i, 0))],
        out_shape=[jax.ShapeDtypeStruct((B * N, D), jnp.float32),
                   jax.ShapeDtypeStruct((B * N, H), jnp.float32)],
    )(x_rows, r1(p['ln1_g']), r1(p['ln1_b']), r1(p['f_n1_g']),
      r1(p['f_n1_b']), fw[2 * D:], r1(fb[2 * D:]), comb, sbias)
    vt3 = vtab.reshape(B, N, D)

    controller = pl.pallas_call(_controller_kernel, out_shape=ctrl_out_shape)
    score_call = pl.pallas_call(
        _score_kernel,
        grid=(ng,),
        in_specs=[pl.BlockSpec((bm, D), lambda i: (i, 0))] +
                 [pl.BlockSpec((1, D), lambda i: (0, 0))] * 4 +
                 [pl.BlockSpec((1, D, H), lambda i: (i // nbb, 0, 0)),
                  pl.BlockSpec((1, 1, H), lambda i: (i // nbb, 0, 0))],
        out_specs=pl.BlockSpec((bm, H), lambda i: (i, 0)),
        out_shape=jax.ShapeDtypeStruct((B * N, H), jnp.float32),
    )

    fps, flogits = [], []
    acc_starts = []
    for t in range(NSACC):
        if t > 0:
            scores, comb, sbias, sst, vs = controller(
                state, pmap, p['c_q_w'], r1(p['c_q_b']), p['c_k_w'],
                r1(p['c_k_b']), r1(p['f_n1_g']), r1(p['f_n1_b']), fw, r1(fb),
                bqc, mask)
            sall = score_call(x_rows, r1(p['ln1_g']), r1(p['ln1_b']),
                              r1(p['f_n1_g']), r1(p['f_n1_b']), comb, sbias)
        ti, tw = _sc_topk_call(scores)
        fps.append(ti[:, 0] * BLK)
        flogits.append(scores)
        starts = jnp.clip(ti * BLK - WS // 2, 0, N - WS) // 8
        sall3 = sall.reshape(B, N, H)

        astack = (jnp.stack(acc_starts, axis=1) if acc_starts
                  else jnp.zeros((B, 1), jnp.int32))
        nacc = len(acc_starts)

        ctxv = pl.pallas_call(
            functools.partial(_foveal_attn_kernel, nacc=nacc),
            grid=(B,),
            in_specs=[pl.BlockSpec(memory_space=pltpu.SMEM),
                      pl.BlockSpec(memory_space=pltpu.SMEM),
                      pl.BlockSpec((1, N, H), lambda b: (b, 0, 0)),
                      pl.BlockSpec((1, N, D), lambda b: (b, 0, 0)),
                      pl.BlockSpec((1, 1, H), lambda b: (b, 0, 0)),
                      pl.BlockSpec((1, 1, D), lambda b: (b, 0, 0)),
                      pl.BlockSpec((H, D), lambda b: (0, 0))],
            out_specs=pl.BlockSpec((1, TOPK, D), lambda b: (b, 0, 0)),
            out_shape=jax.ShapeDtypeStruct((B, TOPK, D), jnp.float32),
        )(starts, astack, sall3, vt3, sst, vs.reshape(B, 1, D), expander)
        ctxv32 = ctxv.reshape(B * TOPK, D)

        acc_starts.append(starts[:, 0])
        astack2 = jnp.stack(acc_starts, axis=1)

        tt = jnp.array([[t / NSACC]], dtype=jnp.float32)
        a1 = _gelu(tt @ p['g1_w'].T + p['g1_b'])
        alpha = jax.nn.sigmoid(a1 @ p['g2_w'].T + p['g2_b'])

        st32 = jnp.repeat(state, TOPK, axis=0)
        state = pl.pallas_call(
            _dense_kernel,
            grid=(4,),
            in_specs=[
                pl.BlockSpec((B * TOPK, D), lambda j: (0, 0)),
                pl.BlockSpec((B * TOPK, D), lambda j: (0, 0)),
                pl.BlockSpec((B, TOPK), lambda j: (0, 0)),
                pl.BlockSpec((D, D), lambda j: (0, 0)),
                pl.BlockSpec((1, D), lambda j: (0, 0)),
                pl.BlockSpec((1, D), lambda j: (0, 0)),
                pl.BlockSpec((1, D), lambda j: (0, 0)),
                pl.BlockSpec((D, D), lambda j: (j, 0)),
                pl.BlockSpec((1, D), lambda j: (0, j)),
                pl.BlockSpec((D, D), lambda j: (0, j)),
                pl.BlockSpec((1, D), lambda j: (0, 0)),
            ],
            out_specs=pl.BlockSpec((B, D), lambda j: (0, 0)),
            out_shape=jax.ShapeDtypeStruct((B, D), jnp.float32),
            scratch_shapes=[pltpu.VMEM((B * TOPK, D), jnp.float32)],
        )(ctxv32, st32, tw, p['f_out_w'], r1(p['f_out_b']),
          r1(p['f_n2_g']), r1(p['f_n2_b']), p['f_ffn1_w'], r1(p['f_ffn1_b']),
          p['f_ffn2_w'], r1(p['f_ffn2_b']))

        pmap = pl.pallas_call(
            functools.partial(_mattn_kernel, nacc=t + 1),
            grid=(3,),
            in_specs=[pl.BlockSpec(memory_space=pltpu.SMEM),
                      pl.BlockSpec(memory_space=pltpu.SMEM),
                      pl.BlockSpec((B, nb, D), lambda j: (0, 0, 0)),
                      pl.BlockSpec((1, D), lambda j: (0, 0)),
                      pl.BlockSpec((1, D), lambda j: (0, 0)),
                      pl.BlockSpec((D, D), lambda j: (j, 0)),
                      pl.BlockSpec((1, D), lambda j: (0, j)),
                      pl.BlockSpec((D, D), lambda j: (0, 0)),
                      pl.BlockSpec((1, D), lambda j: (0, 0)),
                      pl.BlockSpec((1, D), lambda j: (0, 0)),
                      pl.BlockSpec((1, D), lambda j: (0, 0)),
                      pl.BlockSpec(memory_space=pl.ANY)],
            out_specs=pl.BlockSpec((B, nb, D), lambda j: (0, 0, 0)),
            out_shape=jax.ShapeDtypeStruct((B, nb, D), jnp.float32),
            scratch_shapes=[pltpu.VMEM((B, nb, D), jnp.float32),
                            pltpu.VMEM((B, (t + 1) * WS, D), jnp.float32),
                            pltpu.VMEM((B, (t + 1) * WS, D), jnp.float32),
                            pltpu.VMEM((B, (t + 1) * WS, D), jnp.float32),
                            pltpu.SemaphoreType.DMA],
        )(astack2, alpha, pmap, r1(p['m_norm_g']), r1(p['m_norm_b']),
          p['m_in_w'], r1(p['m_in_b']), p['m_out_w'], r1(p['m_out_b']),
          r1(p['ln1_g']), r1(p['ln1_b']), x_sacc)

    bm2 = 512
    out = pl.pallas_call(
        _final_kernel,
        grid=(B, N // bm2),
        in_specs=[
            pl.BlockSpec((1, bm2, D), lambda b, i: (b, i, 0)),
            pl.BlockSpec((1, 1, D), lambda b, i: (b, 0, 0)),
        ] + [pl.BlockSpec((1, D), lambda b, i: (0, 0))] * 2 + [
            pl.BlockSpec((D, D), lambda b, i: (0, 0)),
            pl.BlockSpec((1, D), lambda b, i: (0, 0)),
            pl.BlockSpec((1, D), lambda b, i: (0, 0)),
            pl.BlockSpec((1, D), lambda b, i: (0, 0)),
            pl.BlockSpec((4 * D, D), lambda b, i: (0, 0)),
            pl.BlockSpec((1, 4 * D), lambda b, i: (0, 0)),
            pl.BlockSpec((D, 4 * D), lambda b, i: (0, 0)),
            pl.BlockSpec((1, D), lambda b, i: (0, 0)),
        ],
        out_specs=pl.BlockSpec((1, bm2, D), lambda b, i: (b, i, 0)),
        out_shape=jax.ShapeDtypeStruct((B, N, D), jnp.float32),
    )(x_sacc, state.reshape(B, 1, D), r1(p['o_norm_g']), r1(p['o_norm_b']),
      p['o_w'], r1(p['o_b']), r1(p['ln2_g']), r1(p['ln2_b']),
      p['mlp1_w'].astype(jnp.bfloat16), r1(p['mlp1_b']),
      p['mlp2_w'].astype(jnp.bfloat16), r1(p['mlp2_b']))

    return out, jnp.stack(fps), jnp.stack(flogits)
